# Initial kernel scaffold; baseline (speedup 1.0000x reference)
import jax
import jax.numpy as jnp
from jax.experimental import pallas as pl


def kernel(node_feats, edge_feats, edge_index, W_g1, b_g1, W_g2, b_g2, W_ea, b_ea, W_na, b_na, W_nr, b_nr, W_m1, b_m1, W_m2, b_m2):
    raise NotImplementedError("write your pallas kernel here")


# speedup vs baseline: 2.0633x; 2.0633x over previous
import functools

import jax
import jax.numpy as jnp
from jax import lax
from jax.experimental import pallas as pl
from jax.experimental.pallas import tpu as pltpu

N = 10000
E = 320000
D = 128
H = 64

EB = 2000


def _sigmoid(x):
    return 1.0 / (1.0 + jnp.exp(-x))


def _edge_front_body(ef_ref, w_ref, b_ref, ei_ref, m1_ref):
    p = jnp.dot(ef_ref[...], w_ref[...], preferred_element_type=jnp.float32)
    p = p + b_ref[...]
    ei_ref[...] = p[:, :H]
    m1_ref[...] = jnp.maximum(p[:, H:], 0.0)


def _edge_front(edge_feats, Wcat, bcat):
    grid = (E // EB,)
    return pl.pallas_call(
        _edge_front_body,
        grid=grid,
        in_specs=[
            pl.BlockSpec((EB, D), lambda i: (i, 0)),
            pl.BlockSpec((D, D), lambda i: (0, 0)),
            pl.BlockSpec((1, D), lambda i: (0, 0)),
        ],
        out_specs=[
            pl.BlockSpec((EB, H), lambda i: (i, 0)),
            pl.BlockSpec((EB, H), lambda i: (i, 0)),
        ],
        out_shape=[
            jax.ShapeDtypeStruct((E, H), jnp.float32),
            jax.ShapeDtypeStruct((E, H), jnp.float32),
        ],
    )(edge_feats, Wcat, bcat)


NB = 2000


def _mid_a_body(ss_ref, sd_ref, cs_ref, cd_ref, nf_ref, wg1_ref,
                aei_ref, dinv_ref, xw1_ref, y1_ref):
    cs = cs_ref[0, :, :1] + cs_ref[1, :, :1]
    cd = cd_ref[0, :, :1] + cd_ref[1, :, :1]
    raw_row = (ss_ref[0] + ss_ref[1]) / jnp.maximum(cs, 1.0)
    raw_col = (sd_ref[0] + sd_ref[1]) / jnp.maximum(cd, 1.0)
    aei_ref[...] = _sigmoid(raw_row * raw_col)
    dinv = lax.rsqrt(cd + 1.0)
    dinv_ref[...] = jnp.broadcast_to(dinv, dinv_ref.shape)
    xw1 = jnp.dot(nf_ref[...], wg1_ref[...], preferred_element_type=jnp.float32)
    xw1_ref[...] = xw1
    y1_ref[...] = xw1 * dinv


def _mid_a(sumS, sumD, cS, cD, node_feats, W_g1):
    grid = (N // NB,)
    return pl.pallas_call(
        _mid_a_body,
        grid=grid,
        in_specs=[
            pl.BlockSpec((2, NB, H), lambda i: (0, i, 0)),
            pl.BlockSpec((2, NB, H), lambda i: (0, i, 0)),
            pl.BlockSpec((2, NB, 8), lambda i: (0, i, 0)),
            pl.BlockSpec((2, NB, 8), lambda i: (0, i, 0)),
            pl.BlockSpec((NB, D), lambda i: (i, 0)),
            pl.BlockSpec((D, H), lambda i: (0, 0)),
        ],
        out_specs=[
            pl.BlockSpec((NB, H), lambda i: (i, 0)),
            pl.BlockSpec((NB, 8), lambda i: (i, 0)),
            pl.BlockSpec((NB, H), lambda i: (i, 0)),
            pl.BlockSpec((NB, H), lambda i: (i, 0)),
        ],
        out_shape=[
            jax.ShapeDtypeStruct((N, H), jnp.float32),
            jax.ShapeDtypeStruct((N, 8), jnp.float32),
            jax.ShapeDtypeStruct((N, H), jnp.float32),
            jax.ShapeDtypeStruct((N, H), jnp.float32),
        ],
    )(sumS, sumD, cS, cD, node_feats, W_g1)


def _mid_b_body(agg_ref, xw1_ref, dinv_ref, aei_ref, wg2_ref, bg1_ref,
                xw2_ref, y2_ref):
    dinv = dinv_ref[:, :1]
    h1 = dinv * (agg_ref[0] + agg_ref[1]) + dinv * dinv * xw1_ref[...] + bg1_ref[...]
    x1 = jnp.maximum(h1, 0.0) * aei_ref[...]
    xw2 = jnp.dot(x1, wg2_ref[...], preferred_element_type=jnp.float32)
    xw2_ref[...] = xw2
    y2_ref[...] = xw2 * dinv


def _mid_b(agg1, xw1, dinv8, aei, W_g2, b_g1):
    grid = (N // NB,)
    return pl.pallas_call(
        _mid_b_body,
        grid=grid,
        in_specs=[
            pl.BlockSpec((2, NB, H), lambda i: (0, i, 0)),
            pl.BlockSpec((NB, H), lambda i: (i, 0)),
            pl.BlockSpec((NB, 8), lambda i: (i, 0)),
            pl.BlockSpec((NB, H), lambda i: (i, 0)),
            pl.BlockSpec((H, D), lambda i: (0, 0)),
            pl.BlockSpec((1, H), lambda i: (0, 0)),
        ],
        out_specs=[
            pl.BlockSpec((NB, D), lambda i: (i, 0)),
            pl.BlockSpec((NB, D), lambda i: (i, 0)),
        ],
        out_shape=[
            jax.ShapeDtypeStruct((N, D), jnp.float32),
            jax.ShapeDtypeStruct((N, D), jnp.float32),
        ],
    )(agg1, xw1, dinv8, aei, W_g2, b_g1)


def _mid_c_body(agg_ref, xw2_ref, dinv_ref, wna_ref, bna_ref,
                wnrt_ref, wnrb_ref, bnr_ref, bg2_ref,
                x_ref, a_ref, b_ref):
    dinv = dinv_ref[:, :1]
    h2 = dinv * (agg_ref[0] + agg_ref[1]) + dinv * dinv * xw2_ref[...] + bg2_ref[...]
    x = jnp.maximum(h2, 0.0)
    x_ref[...] = x
    ni = jnp.maximum(
        jnp.dot(x, wna_ref[...], preferred_element_type=jnp.float32) + bna_ref[...],
        0.0)
    a_ref[...] = jnp.dot(ni, wnrt_ref[...], preferred_element_type=jnp.float32) + bnr_ref[...]
    b_ref[...] = jnp.dot(ni, wnrb_ref[...], preferred_element_type=jnp.float32)


def _mid_c(agg2, xw2, dinv8, W_na, b_na, W_nr_t, W_nr_b, b_nr, b_g2):
    grid = (N // NB,)
    return pl.pallas_call(
        _mid_c_body,
        grid=grid,
        in_specs=[
            pl.BlockSpec((2, NB, D), lambda i: (0, i, 0)),
            pl.BlockSpec((NB, D), lambda i: (i, 0)),
            pl.BlockSpec((NB, 8), lambda i: (i, 0)),
            pl.BlockSpec((D, H), lambda i: (0, 0)),
            pl.BlockSpec((1, H), lambda i: (0, 0)),
            pl.BlockSpec((H, H), lambda i: (0, 0)),
            pl.BlockSpec((H, H), lambda i: (0, 0)),
            pl.BlockSpec((1, H), lambda i: (0, 0)),
            pl.BlockSpec((1, D), lambda i: (0, 0)),
        ],
        out_specs=[
            pl.BlockSpec((NB, D), lambda i: (i, 0)),
            pl.BlockSpec((NB, H), lambda i: (i, 0)),
            pl.BlockSpec((NB, H), lambda i: (i, 0)),
        ],
        out_shape=[
            jax.ShapeDtypeStruct((N, D), jnp.float32),
            jax.ShapeDtypeStruct((N, H), jnp.float32),
            jax.ShapeDtypeStruct((N, H), jnp.float32),
        ],
    )(agg2, xw2, dinv8, W_na, b_na, W_nr_t, W_nr_b, b_nr, b_g2)


def _edge_back_body(m1_ref, ra_ref, rb_ref, wm2_ref, bm2_ref, e_ref):
    g = m1_ref[...] * _sigmoid(ra_ref[...] + rb_ref[...])
    e = jnp.dot(g, wm2_ref[...], preferred_element_type=jnp.float32) + bm2_ref[...]
    e_ref[...] = jnp.maximum(e, 0.0)


def _edge_back(m1, rA, rB, W_m2, b_m2):
    grid = (E // EB,)
    return pl.pallas_call(
        _edge_back_body,
        grid=grid,
        in_specs=[
            pl.BlockSpec((EB, H), lambda i: (i, 0)),
            pl.BlockSpec((EB, H), lambda i: (i, 0)),
            pl.BlockSpec((EB, H), lambda i: (i, 0)),
            pl.BlockSpec((H, D), lambda i: (0, 0)),
            pl.BlockSpec((1, D), lambda i: (0, 0)),
        ],
        out_specs=pl.BlockSpec((EB, D), lambda i: (i, 0)),
        out_shape=jax.ShapeDtypeStruct((E, D), jnp.float32),
    )(m1, rA, rB, W_m2, b_m2)


def kernel(node_feats, edge_feats, edge_index, W_g1, b_g1, W_g2, b_g2,
           W_ea, b_ea, W_na, b_na, W_nr, b_nr, W_m1, b_m1, W_m2, b_m2):
    src = edge_index[0]
    dst = edge_index[1]

    Wcat = jnp.concatenate([W_ea, W_m1], axis=1)
    bcat = jnp.concatenate([b_ea, b_m1])[None, :]
    edge_ind, m1 = _edge_front(edge_feats, Wcat, bcat)

    def seg_sum(vals, idx):
        return jnp.zeros((N, vals.shape[1]), vals.dtype).at[idx].add(vals)

    cnt_src = jnp.zeros((N,), jnp.float32).at[src].add(1.0)
    cnt_dst = jnp.zeros((N,), jnp.float32).at[dst].add(1.0)
    sumS = jnp.stack([seg_sum(edge_ind, src), jnp.zeros((N, H))])
    sumD = jnp.stack([seg_sum(edge_ind, dst), jnp.zeros((N, H))])
    cS = jnp.stack([jnp.broadcast_to(cnt_src[:, None], (N, 8)), jnp.zeros((N, 8))])
    cD = jnp.stack([jnp.broadcast_to(cnt_dst[:, None], (N, 8)), jnp.zeros((N, 8))])

    aei, dinv8, xw1, y1 = _mid_a(sumS, sumD, cS, cD, node_feats, W_g1)

    agg1 = jnp.stack([seg_sum(y1[src], dst), jnp.zeros((N, H))])
    xw2, y2 = _mid_b(agg1, xw1, dinv8, aei, W_g2, b_g1[None, :])

    agg2 = jnp.stack([seg_sum(y2[src], dst), jnp.zeros((N, D))])
    x, A, B = _mid_c(agg2, xw2, dinv8, W_na, b_na[None, :],
                     W_nr[:H], W_nr[H:], b_nr[None, :], b_g2[None, :])

    rA = A[src]
    rB = B[dst]
    e = _edge_back(m1, rA, rB, W_m2, b_m2[None, :])
    return (x, e)


# speedup vs baseline: 8.3142x; 4.0296x over previous
import functools

import jax
import jax.numpy as jnp
from jax import lax
from jax.experimental import pallas as pl
from jax.experimental.pallas import tpu as pltpu
from jax.experimental.pallas import tpu_sc as plsc

N = 10000
E = 320000
D = 128
H = 64

EB = 2000

NC, NS = 2, 16
NW = NC * NS
U = 80
UPC = 5
CE = U * UPC
NUNITS = E // U
UPW = NUNITS // NW
CPW = UPW // UPC
NPAD = N
RPS = NPAD // NS

_SC_PARAMS = pltpu.CompilerParams(use_tc_tiling_on_sc=False)

_SC_MESH = plsc.VectorSubcoreMesh(
    core_axis_name="c", subcore_axis_name="s", num_cores=NC, num_subcores=NS)


def _sigmoid(x):
    return 1.0 / (1.0 + jnp.exp(-x))


def _worker_coords():
    cid = lax.axis_index("c")
    sid = lax.axis_index("s")
    return cid, sid, sid * NC + cid


def _sc_scatter_counts_body(src2_hbm, dst2_hbm, ei_hbm, z64_hbm, z8_hbm, o8_hbm,
                            outS_hbm, outD_hbm, outCS_hbm, outCD_hbm,
                            sidx2, didx2, vals_v, ones_v,
                            accS, accD, cS8, cD8, sem):
    cid, sid, w = _worker_coords()
    lo = sid * RPS
    pltpu.sync_copy(z64_hbm, accS.at[pl.ds(lo, RPS)])
    pltpu.sync_copy(z64_hbm, accD.at[pl.ds(lo, RPS)])
    pltpu.sync_copy(z8_hbm, cS8.at[pl.ds(lo, RPS)])
    pltpu.sync_copy(z8_hbm, cD8.at[pl.ds(lo, RPS)])
    pltpu.sync_copy(o8_hbm, ones_v)
    plsc.subcore_barrier()

    def chunk(c, carry):
        ub = w * UPW + c * UPC
        pltpu.sync_copy(src2_hbm.at[pl.ds(ub, UPC)], sidx2)
        pltpu.sync_copy(dst2_hbm.at[pl.ds(ub, UPC)], didx2)
        pltpu.sync_copy(ei_hbm.at[pl.ds(ub * U, CE)], vals_v)
        cps = []
        for j in range(UPC):
            v = vals_v.at[pl.ds(j * U, U)]
            cps.append(pltpu.async_copy(v, accS.at[sidx2.at[j]], sem, add=True))
            cps.append(pltpu.async_copy(v, accD.at[didx2.at[j]], sem, add=True))
            cps.append(pltpu.async_copy(ones_v, cS8.at[sidx2.at[j]], sem, add=True))
            cps.append(pltpu.async_copy(ones_v, cD8.at[didx2.at[j]], sem, add=True))
        for cp in cps:
            cp.wait()
        return carry

    lax.fori_loop(0, CPW, chunk, 0)
    plsc.subcore_barrier()
    pltpu.sync_copy(accS.at[pl.ds(lo, RPS)], outS_hbm.at[cid, pl.ds(lo, RPS)])
    pltpu.sync_copy(accD.at[pl.ds(lo, RPS)], outD_hbm.at[cid, pl.ds(lo, RPS)])
    pltpu.sync_copy(cS8.at[pl.ds(lo, RPS)], outCS_hbm.at[cid, pl.ds(lo, RPS)])
    pltpu.sync_copy(cD8.at[pl.ds(lo, RPS)], outCD_hbm.at[cid, pl.ds(lo, RPS)])


def _sc_scatter_counts(src2, dst2, edge_ind):
    z64 = jnp.zeros((RPS, H), jnp.float32)
    z8 = jnp.zeros((RPS, 8), jnp.float32)
    o8 = jnp.ones((U, 8), jnp.float32)
    fn = pl.kernel(
        _sc_scatter_counts_body,
        out_type=[
            jax.ShapeDtypeStruct((NC, NPAD, H), jnp.float32),
            jax.ShapeDtypeStruct((NC, NPAD, H), jnp.float32),
            jax.ShapeDtypeStruct((NC, NPAD, 8), jnp.float32),
            jax.ShapeDtypeStruct((NC, NPAD, 8), jnp.float32),
        ],
        mesh=_SC_MESH,
        compiler_params=_SC_PARAMS,
        scratch_types=[
            pltpu.VMEM((UPC, U), jnp.int32),
            pltpu.VMEM((UPC, U), jnp.int32),
            pltpu.VMEM((CE, H), jnp.float32),
            pltpu.VMEM((U, 8), jnp.float32),
            pltpu.VMEM_SHARED((NPAD, H), jnp.float32),
            pltpu.VMEM_SHARED((NPAD, H), jnp.float32),
            pltpu.VMEM_SHARED((NPAD, 8), jnp.float32),
            pltpu.VMEM_SHARED((NPAD, 8), jnp.float32),
            pltpu.SemaphoreType.DMA,
        ],
    )
    return fn(src2, dst2, edge_ind, z64, z8, o8)


def _sc_gcn_agg_body(src2_hbm, dst2_hbm, y_hbm, zF_hbm, out_hbm,
                     sidx2, didx2, vals_v, acc, semg, sems):
    cid, sid, w = _worker_coords()
    lo = sid * RPS
    pltpu.sync_copy(zF_hbm, acc.at[pl.ds(lo, RPS)])
    plsc.subcore_barrier()

    def chunk(c, carry):
        ub = w * UPW + c * UPC
        pltpu.sync_copy(src2_hbm.at[pl.ds(ub, UPC)], sidx2)
        pltpu.sync_copy(dst2_hbm.at[pl.ds(ub, UPC)], didx2)
        gcps = [pltpu.async_copy(y_hbm.at[sidx2.at[j]],
                                 vals_v.at[pl.ds(j * U, U)], semg)
                for j in range(UPC)]
        scps = []
        for j in range(UPC):
            gcps[j].wait()
            scps.append(pltpu.async_copy(vals_v.at[pl.ds(j * U, U)],
                                         acc.at[didx2.at[j]], sems, add=True))
        for cp in scps:
            cp.wait()
        return carry

    lax.fori_loop(0, CPW, chunk, 0)
    plsc.subcore_barrier()
    pltpu.sync_copy(acc.at[pl.ds(lo, RPS)], out_hbm.at[cid, pl.ds(lo, RPS)])


def _sc_gcn_agg(src2, dst2, y, F):
    zF = jnp.zeros((RPS, F), jnp.float32)
    fn = pl.kernel(
        _sc_gcn_agg_body,
        out_type=jax.ShapeDtypeStruct((NC, NPAD, F), jnp.float32),
        mesh=_SC_MESH,
        compiler_params=_SC_PARAMS,
        scratch_types=[
            pltpu.VMEM((UPC, U), jnp.int32),
            pltpu.VMEM((UPC, U), jnp.int32),
            pltpu.VMEM((CE, F), jnp.float32),
            pltpu.VMEM_SHARED((NPAD, F), jnp.float32),
            pltpu.SemaphoreType.DMA,
            pltpu.SemaphoreType.DMA,
        ],
    )
    return fn(src2, dst2, y, zF)


def _sc_edge_gather_body(src2_hbm, dst2_hbm, a_hbm, b_hbm,
                         outA_hbm, outB_hbm,
                         sidx2, didx2, valsA, valsB, semg):
    cid, sid, w = _worker_coords()

    def chunk(c, carry):
        ub = w * UPW + c * UPC
        pltpu.sync_copy(src2_hbm.at[pl.ds(ub, UPC)], sidx2)
        pltpu.sync_copy(dst2_hbm.at[pl.ds(ub, UPC)], didx2)
        acps = [pltpu.async_copy(a_hbm.at[sidx2.at[j]],
                                 valsA.at[pl.ds(j * U, U)], semg)
                for j in range(UPC)]
        bcps = [pltpu.async_copy(b_hbm.at[didx2.at[j]],
                                 valsB.at[pl.ds(j * U, U)], semg)
                for j in range(UPC)]
        for cp in acps:
            cp.wait()
        pltpu.sync_copy(valsA, outA_hbm.at[pl.ds(ub * U, CE)])
        for cp in bcps:
            cp.wait()
        pltpu.sync_copy(valsB, outB_hbm.at[pl.ds(ub * U, CE)])
        return carry

    lax.fori_loop(0, CPW, chunk, 0)


def _sc_edge_gather(src2, dst2, A, B):
    fn = pl.kernel(
        _sc_edge_gather_body,
        out_type=[
            jax.ShapeDtypeStruct((E, H), jnp.float32),
            jax.ShapeDtypeStruct((E, H), jnp.float32),
        ],
        mesh=_SC_MESH,
        compiler_params=_SC_PARAMS,
        scratch_types=[
            pltpu.VMEM((UPC, U), jnp.int32),
            pltpu.VMEM((UPC, U), jnp.int32),
            pltpu.VMEM((CE, H), jnp.float32),
            pltpu.VMEM((CE, H), jnp.float32),
            pltpu.SemaphoreType.DMA,
        ],
    )
    return fn(src2, dst2, A, B)


def _edge_front_body(ef_ref, w_ref, b_ref, ei_ref, m1_ref):
    p = jnp.dot(ef_ref[...], w_ref[...], preferred_element_type=jnp.float32)
    p = p + b_ref[...]
    ei_ref[...] = p[:, :H]
    m1_ref[...] = jnp.maximum(p[:, H:], 0.0)


def _edge_front(edge_feats, Wcat, bcat):
    grid = (E // EB,)
    return pl.pallas_call(
        _edge_front_body,
        grid=grid,
        in_specs=[
            pl.BlockSpec((EB, D), lambda i: (i, 0)),
            pl.BlockSpec((D, D), lambda i: (0, 0)),
            pl.BlockSpec((1, D), lambda i: (0, 0)),
        ],
        out_specs=[
            pl.BlockSpec((EB, H), lambda i: (i, 0)),
            pl.BlockSpec((EB, H), lambda i: (i, 0)),
        ],
        out_shape=[
            jax.ShapeDtypeStruct((E, H), jnp.float32),
            jax.ShapeDtypeStruct((E, H), jnp.float32),
        ],
    )(edge_feats, Wcat, bcat)


NB = 2000


def _mid_a_body(ss_ref, sd_ref, cs_ref, cd_ref, nf_ref, wg1_ref,
                aei_ref, dinv_ref, xw1_ref, y1_ref):
    cs = cs_ref[0, :, :1] + cs_ref[1, :, :1]
    cd = cd_ref[0, :, :1] + cd_ref[1, :, :1]
    raw_row = (ss_ref[0] + ss_ref[1]) / jnp.maximum(cs, 1.0)
    raw_col = (sd_ref[0] + sd_ref[1]) / jnp.maximum(cd, 1.0)
    aei_ref[...] = _sigmoid(raw_row * raw_col)
    dinv = lax.rsqrt(cd + 1.0)
    dinv_ref[...] = jnp.broadcast_to(dinv, dinv_ref.shape)
    xw1 = jnp.dot(nf_ref[...], wg1_ref[...], preferred_element_type=jnp.float32)
    xw1_ref[...] = xw1
    y1_ref[...] = xw1 * dinv


def _mid_a(sumS, sumD, cS, cD, node_feats, W_g1):
    grid = (NPAD // NB,)
    return pl.pallas_call(
        _mid_a_body,
        grid=grid,
        in_specs=[
            pl.BlockSpec((2, NB, H), lambda i: (0, i, 0)),
            pl.BlockSpec((2, NB, H), lambda i: (0, i, 0)),
            pl.BlockSpec((2, NB, 8), lambda i: (0, i, 0)),
            pl.BlockSpec((2, NB, 8), lambda i: (0, i, 0)),
            pl.BlockSpec((NB, D), lambda i: (i, 0)),
            pl.BlockSpec((D, H), lambda i: (0, 0)),
        ],
        out_specs=[
            pl.BlockSpec((NB, H), lambda i: (i, 0)),
            pl.BlockSpec((NB, 8), lambda i: (i, 0)),
            pl.BlockSpec((NB, H), lambda i: (i, 0)),
            pl.BlockSpec((NB, H), lambda i: (i, 0)),
        ],
        out_shape=[
            jax.ShapeDtypeStruct((NPAD, H), jnp.float32),
            jax.ShapeDtypeStruct((NPAD, 8), jnp.float32),
            jax.ShapeDtypeStruct((NPAD, H), jnp.float32),
            jax.ShapeDtypeStruct((NPAD, H), jnp.float32),
        ],
    )(sumS, sumD, cS, cD, node_feats, W_g1)


def _mid_b_body(agg_ref, xw1_ref, dinv_ref, aei_ref, wg2_ref, bg1_ref,
                xw2_ref, y2a_ref, y2b_ref):
    dinv = dinv_ref[:, :1]
    h1 = dinv * (agg_ref[0] + agg_ref[1]) + dinv * dinv * xw1_ref[...] + bg1_ref[...]
    x1 = jnp.maximum(h1, 0.0) * aei_ref[...]
    xw2 = jnp.dot(x1, wg2_ref[...], preferred_element_type=jnp.float32)
    xw2_ref[...] = xw2
    y2 = xw2 * dinv
    y2a_ref[...] = y2[:, :H]
    y2b_ref[...] = y2[:, H:]


def _mid_b(agg1, xw1, dinv8, aei, W_g2, b_g1):
    grid = (NPAD // NB,)
    return pl.pallas_call(
        _mid_b_body,
        grid=grid,
        in_specs=[
            pl.BlockSpec((2, NB, H), lambda i: (0, i, 0)),
            pl.BlockSpec((NB, H), lambda i: (i, 0)),
            pl.BlockSpec((NB, 8), lambda i: (i, 0)),
            pl.BlockSpec((NB, H), lambda i: (i, 0)),
            pl.BlockSpec((H, D), lambda i: (0, 0)),
            pl.BlockSpec((1, H), lambda i: (0, 0)),
        ],
        out_specs=[
            pl.BlockSpec((NB, D), lambda i: (i, 0)),
            pl.BlockSpec((NB, H), lambda i: (i, 0)),
            pl.BlockSpec((NB, H), lambda i: (i, 0)),
        ],
        out_shape=[
            jax.ShapeDtypeStruct((NPAD, D), jnp.float32),
            jax.ShapeDtypeStruct((NPAD, H), jnp.float32),
            jax.ShapeDtypeStruct((NPAD, H), jnp.float32),
        ],
    )(agg1, xw1, dinv8, aei, W_g2, b_g1)


def _mid_c_body(agga_ref, aggb_ref, xw2_ref, dinv_ref, wna_ref, bna_ref,
                wnrt_ref, wnrb_ref, bnr_ref, bg2_ref,
                x_ref, a_ref, b_ref):
    dinv = dinv_ref[:, :1]
    aggtot = jnp.concatenate(
        [agga_ref[0] + agga_ref[1], aggb_ref[0] + aggb_ref[1]], axis=-1)
    h2 = dinv * aggtot + dinv * dinv * xw2_ref[...] + bg2_ref[...]
    x = jnp.maximum(h2, 0.0)
    x_ref[...] = x
    ni = jnp.maximum(
        jnp.dot(x, wna_ref[...], preferred_element_type=jnp.float32) + bna_ref[...],
        0.0)
    a_ref[...] = jnp.dot(ni, wnrt_ref[...], preferred_element_type=jnp.float32) + bnr_ref[...]
    b_ref[...] = jnp.dot(ni, wnrb_ref[...], preferred_element_type=jnp.float32)


def _mid_c(agg2a, agg2b, xw2, dinv8, W_na, b_na, W_nr_t, W_nr_b, b_nr, b_g2):
    grid = (NPAD // NB,)
    return pl.pallas_call(
        _mid_c_body,
        grid=grid,
        in_specs=[
            pl.BlockSpec((2, NB, H), lambda i: (0, i, 0)),
            pl.BlockSpec((2, NB, H), lambda i: (0, i, 0)),
            pl.BlockSpec((NB, D), lambda i: (i, 0)),
            pl.BlockSpec((NB, 8), lambda i: (i, 0)),
            pl.BlockSpec((D, H), lambda i: (0, 0)),
            pl.BlockSpec((1, H), lambda i: (0, 0)),
            pl.BlockSpec((H, H), lambda i: (0, 0)),
            pl.BlockSpec((H, H), lambda i: (0, 0)),
            pl.BlockSpec((1, H), lambda i: (0, 0)),
            pl.BlockSpec((1, D), lambda i: (0, 0)),
        ],
        out_specs=[
            pl.BlockSpec((NB, D), lambda i: (i, 0)),
            pl.BlockSpec((NB, H), lambda i: (i, 0)),
            pl.BlockSpec((NB, H), lambda i: (i, 0)),
        ],
        out_shape=[
            jax.ShapeDtypeStruct((NPAD, D), jnp.float32),
            jax.ShapeDtypeStruct((NPAD, H), jnp.float32),
            jax.ShapeDtypeStruct((NPAD, H), jnp.float32),
        ],
    )(agg2a, agg2b, xw2, dinv8, W_na, b_na, W_nr_t, W_nr_b, b_nr, b_g2)


def _edge_back_body(m1_ref, ra_ref, rb_ref, wm2_ref, bm2_ref, e_ref):
    g = m1_ref[...] * _sigmoid(ra_ref[...] + rb_ref[...])
    e = jnp.dot(g, wm2_ref[...], preferred_element_type=jnp.float32) + bm2_ref[...]
    e_ref[...] = jnp.maximum(e, 0.0)


def _edge_back(m1, rA, rB, W_m2, b_m2):
    grid = (E // EB,)
    return pl.pallas_call(
        _edge_back_body,
        grid=grid,
        in_specs=[
            pl.BlockSpec((EB, H), lambda i: (i, 0)),
            pl.BlockSpec((EB, H), lambda i: (i, 0)),
            pl.BlockSpec((EB, H), lambda i: (i, 0)),
            pl.BlockSpec((H, D), lambda i: (0, 0)),
            pl.BlockSpec((1, D), lambda i: (0, 0)),
        ],
        out_specs=pl.BlockSpec((EB, D), lambda i: (i, 0)),
        out_shape=jax.ShapeDtypeStruct((E, D), jnp.float32),
    )(m1, rA, rB, W_m2, b_m2)


def kernel(node_feats, edge_feats, edge_index, W_g1, b_g1, W_g2, b_g2,
           W_ea, b_ea, W_na, b_na, W_nr, b_nr, W_m1, b_m1, W_m2, b_m2):
    src2 = edge_index[0].reshape(NUNITS, U)
    dst2 = edge_index[1].reshape(NUNITS, U)
    nf_pad = jnp.pad(node_feats, ((0, NPAD - N), (0, 0)))

    Wcat = jnp.concatenate([W_ea, W_m1], axis=1)
    bcat = jnp.concatenate([b_ea, b_m1])[None, :]
    edge_ind, m1 = _edge_front(edge_feats, Wcat, bcat)

    src = edge_index[0]
    dst = edge_index[1]

    def _xla_seg(vals, idx):
        z = jnp.zeros((N, vals.shape[1]), vals.dtype).at[idx].add(vals)
        return jnp.stack([z, jnp.zeros_like(z)])

    USE_SC = {"A": True, "B": True, "C": True, "D": True}

    if USE_SC["A"]:
        sumS, sumD, cS, cD = _sc_scatter_counts(src2, dst2, edge_ind)
    else:
        cnt_s = jnp.zeros((N,), jnp.float32).at[src].add(1.0)
        cnt_d = jnp.zeros((N,), jnp.float32).at[dst].add(1.0)
        sumS = _xla_seg(edge_ind, src)
        sumD = _xla_seg(edge_ind, dst)
        cS = jnp.stack([jnp.broadcast_to(cnt_s[:, None], (N, 8)), jnp.zeros((N, 8))])
        cD = jnp.stack([jnp.broadcast_to(cnt_d[:, None], (N, 8)), jnp.zeros((N, 8))])
    aei, dinv8, xw1, y1 = _mid_a(sumS, sumD, cS, cD, nf_pad, W_g1)

    agg1 = (_sc_gcn_agg(src2, dst2, y1, H) if USE_SC["B"]
            else _xla_seg(y1[src], dst))
    xw2, y2a, y2b = _mid_b(agg1, xw1, dinv8, aei, W_g2, b_g1[None, :])

    if USE_SC["C"]:
        agg2a = _sc_gcn_agg(src2, dst2, y2a, H)
        agg2b = _sc_gcn_agg(src2, dst2, y2b, H)
    else:
        agg2a = _xla_seg(y2a[src], dst)
        agg2b = _xla_seg(y2b[src], dst)
    x, A, B = _mid_c(agg2a, agg2b, xw2, dinv8, W_na, b_na[None, :],
                     W_nr[:H], W_nr[H:], b_nr[None, :], b_g2[None, :])

    if USE_SC["D"]:
        rA, rB = _sc_edge_gather(src2, dst2, A, B)
    else:
        rA, rB = A[src], B[dst]
    e = _edge_back(m1, rA, rB, W_m2, b_m2[None, :])
    return (x[:N], e)


# speedup vs baseline: 8.8466x; 1.0640x over previous
import functools

import jax
import jax.numpy as jnp
from jax import lax
from jax.experimental import pallas as pl
from jax.experimental.pallas import tpu as pltpu
from jax.experimental.pallas import tpu_sc as plsc

N = 10000
E = 320000
D = 128
H = 64

EB = 2000

NC, NS = 2, 16
NW = NC * NS
U = 80
UPC = 5
CE = U * UPC
NUNITS = E // U
UPW = NUNITS // NW
CPW = UPW // UPC
NPAD = N
RPS = NPAD // NS

_SC_PARAMS = pltpu.CompilerParams(use_tc_tiling_on_sc=False)

_SC_MESH = plsc.VectorSubcoreMesh(
    core_axis_name="c", subcore_axis_name="s", num_cores=NC, num_subcores=NS)


def _sigmoid(x):
    return 1.0 / (1.0 + jnp.exp(-x))


def _worker_coords():
    cid = lax.axis_index("c")
    sid = lax.axis_index("s")
    return cid, sid, sid * NC + cid


def _sc_scatter_counts_body(src2_hbm, dst2_hbm, ei_hbm, z64_hbm, z8_hbm, o8_hbm,
                            outS_hbm, outD_hbm, outCS_hbm, outCD_hbm,
                            sidx2, didx2, vals_v, ones_v,
                            accS, accD, cS8, cD8, sem):
    cid, sid, w = _worker_coords()
    lo = sid * RPS
    pltpu.sync_copy(z64_hbm, accS.at[pl.ds(lo, RPS)])
    pltpu.sync_copy(z64_hbm, accD.at[pl.ds(lo, RPS)])
    pltpu.sync_copy(z8_hbm, cS8.at[pl.ds(lo, RPS)])
    pltpu.sync_copy(z8_hbm, cD8.at[pl.ds(lo, RPS)])
    pltpu.sync_copy(o8_hbm, ones_v)
    plsc.subcore_barrier()

    def chunk(c, carry):
        ub = w * UPW + c * UPC
        pltpu.sync_copy(src2_hbm.at[pl.ds(ub, UPC)], sidx2)
        pltpu.sync_copy(dst2_hbm.at[pl.ds(ub, UPC)], didx2)
        pltpu.sync_copy(ei_hbm.at[pl.ds(ub * U, CE)], vals_v)
        cps = []
        for j in range(UPC):
            v = vals_v.at[pl.ds(j * U, U)]
            cps.append(pltpu.async_copy(v, accS.at[sidx2.at[j]], sem, add=True))
            cps.append(pltpu.async_copy(v, accD.at[didx2.at[j]], sem, add=True))
            cps.append(pltpu.async_copy(ones_v, cS8.at[sidx2.at[j]], sem, add=True))
            cps.append(pltpu.async_copy(ones_v, cD8.at[didx2.at[j]], sem, add=True))
        for cp in cps:
            cp.wait()
        return carry

    lax.fori_loop(0, CPW, chunk, 0)
    plsc.subcore_barrier()
    pltpu.sync_copy(accS.at[pl.ds(lo, RPS)], outS_hbm.at[cid, pl.ds(lo, RPS)])
    pltpu.sync_copy(accD.at[pl.ds(lo, RPS)], outD_hbm.at[cid, pl.ds(lo, RPS)])
    pltpu.sync_copy(cS8.at[pl.ds(lo, RPS)], outCS_hbm.at[cid, pl.ds(lo, RPS)])
    pltpu.sync_copy(cD8.at[pl.ds(lo, RPS)], outCD_hbm.at[cid, pl.ds(lo, RPS)])


def _sc_scatter_counts(src2, dst2, edge_ind):
    z64 = jnp.zeros((RPS, H), jnp.float32)
    z8 = jnp.zeros((RPS, 8), jnp.float32)
    o8 = jnp.ones((U, 8), jnp.float32)
    fn = pl.kernel(
        _sc_scatter_counts_body,
        out_type=[
            jax.ShapeDtypeStruct((NC, NPAD, H), jnp.float32),
            jax.ShapeDtypeStruct((NC, NPAD, H), jnp.float32),
            jax.ShapeDtypeStruct((NC, NPAD, 8), jnp.float32),
            jax.ShapeDtypeStruct((NC, NPAD, 8), jnp.float32),
        ],
        mesh=_SC_MESH,
        compiler_params=_SC_PARAMS,
        scratch_types=[
            pltpu.VMEM((UPC, U), jnp.int32),
            pltpu.VMEM((UPC, U), jnp.int32),
            pltpu.VMEM((CE, H), jnp.float32),
            pltpu.VMEM((U, 8), jnp.float32),
            pltpu.VMEM_SHARED((NPAD, H), jnp.float32),
            pltpu.VMEM_SHARED((NPAD, H), jnp.float32),
            pltpu.VMEM_SHARED((NPAD, 8), jnp.float32),
            pltpu.VMEM_SHARED((NPAD, 8), jnp.float32),
            pltpu.SemaphoreType.DMA,
        ],
    )
    return fn(src2, dst2, edge_ind, z64, z8, o8)


def _sc_gcn_agg_body(src2_hbm, dst2_hbm, y_hbm, zF_hbm, out_hbm,
                     sidx, didx, vals0, vals1, acc, semg, sems0, sems1):
    cid, sid, w = _worker_coords()
    lo = sid * RPS
    pltpu.sync_copy(zF_hbm, acc.at[pl.ds(lo, RPS)])
    pltpu.sync_copy(src2_hbm.at[pl.ds(w * UPW, UPW)], sidx)
    pltpu.sync_copy(dst2_hbm.at[pl.ds(w * UPW, UPW)], didx)
    plsc.subcore_barrier()

    bufs = (vals0, vals1)
    sems = (sems0, sems1)
    pend = [[], []]
    for c in range(CPW):
        b = c % 2
        buf = bufs[b]
        for cp in pend[b]:
            cp.wait()
        pend[b] = []
        gcps = [pltpu.async_copy(y_hbm.at[sidx.at[c * UPC + j]],
                                 buf.at[pl.ds(j * U, U)], semg)
                for j in range(UPC)]
        for j in range(UPC):
            gcps[j].wait()
            pend[b].append(pltpu.async_copy(buf.at[pl.ds(j * U, U)],
                                            acc.at[didx.at[c * UPC + j]],
                                            sems[b], add=True))
    for lst in pend:
        for cp in lst:
            cp.wait()
    plsc.subcore_barrier()
    pltpu.sync_copy(acc.at[pl.ds(lo, RPS)], out_hbm.at[cid, pl.ds(lo, RPS)])


def _sc_gcn_agg(src2, dst2, y, F):
    zF = jnp.zeros((RPS, F), jnp.float32)
    fn = pl.kernel(
        _sc_gcn_agg_body,
        out_type=jax.ShapeDtypeStruct((NC, NPAD, F), jnp.float32),
        mesh=_SC_MESH,
        compiler_params=_SC_PARAMS,
        scratch_types=[
            pltpu.VMEM((UPW, U), jnp.int32),
            pltpu.VMEM((UPW, U), jnp.int32),
            pltpu.VMEM((CE, F), jnp.float32),
            pltpu.VMEM((CE, F), jnp.float32),
            pltpu.VMEM_SHARED((NPAD, F), jnp.float32),
            pltpu.SemaphoreType.DMA,
            pltpu.SemaphoreType.DMA,
            pltpu.SemaphoreType.DMA,
        ],
    )
    return fn(src2, dst2, y, zF)


def _sc_edge_gather_body(src2_hbm, dst2_hbm, a_hbm, b_hbm,
                         outA_hbm, outB_hbm,
                         sidx, didx, vA0, vA1, vB0, vB1, semg, semst0, semst1):
    cid, sid, w = _worker_coords()
    pltpu.sync_copy(src2_hbm.at[pl.ds(w * UPW, UPW)], sidx)
    pltpu.sync_copy(dst2_hbm.at[pl.ds(w * UPW, UPW)], didx)

    bufsA = (vA0, vA1)
    bufsB = (vB0, vB1)
    semst = (semst0, semst1)
    pend = [[], []]
    for c in range(CPW):
        b = c % 2
        ub = w * UPW + c * UPC
        for cp in pend[b]:
            cp.wait()
        pend[b] = []
        gcps = [pltpu.async_copy(a_hbm.at[sidx.at[c * UPC + j]],
                                 bufsA[b].at[pl.ds(j * U, U)], semg)
                for j in range(UPC)]
        gcps += [pltpu.async_copy(b_hbm.at[didx.at[c * UPC + j]],
                                  bufsB[b].at[pl.ds(j * U, U)], semg)
                 for j in range(UPC)]
        for cp in gcps:
            cp.wait()
        pend[b].append(pltpu.async_copy(
            bufsA[b], outA_hbm.at[pl.ds(ub * U, CE)], semst[b]))
        pend[b].append(pltpu.async_copy(
            bufsB[b], outB_hbm.at[pl.ds(ub * U, CE)], semst[b]))
    for lst in pend:
        for cp in lst:
            cp.wait()


def _sc_edge_gather(src2, dst2, A, B):
    fn = pl.kernel(
        _sc_edge_gather_body,
        out_type=[
            jax.ShapeDtypeStruct((E, H), jnp.float32),
            jax.ShapeDtypeStruct((E, H), jnp.float32),
        ],
        mesh=_SC_MESH,
        compiler_params=_SC_PARAMS,
        scratch_types=[
            pltpu.VMEM((UPW, U), jnp.int32),
            pltpu.VMEM((UPW, U), jnp.int32),
            pltpu.VMEM((CE, H), jnp.float32),
            pltpu.VMEM((CE, H), jnp.float32),
            pltpu.VMEM((CE, H), jnp.float32),
            pltpu.VMEM((CE, H), jnp.float32),
            pltpu.SemaphoreType.DMA,
            pltpu.SemaphoreType.DMA,
            pltpu.SemaphoreType.DMA,
        ],
    )
    return fn(src2, dst2, A, B)


def _edge_front_body(ef_ref, w_ref, b_ref, ei_ref, m1_ref):
    p = jnp.dot(ef_ref[...], w_ref[...], preferred_element_type=jnp.float32)
    p = p + b_ref[...]
    ei_ref[...] = p[:, :H]
    m1_ref[...] = jnp.maximum(p[:, H:], 0.0)


def _edge_front(edge_feats, Wcat, bcat):
    grid = (E // EB,)
    return pl.pallas_call(
        _edge_front_body,
        grid=grid,
        in_specs=[
            pl.BlockSpec((EB, D), lambda i: (i, 0)),
            pl.BlockSpec((D, D), lambda i: (0, 0)),
            pl.BlockSpec((1, D), lambda i: (0, 0)),
        ],
        out_specs=[
            pl.BlockSpec((EB, H), lambda i: (i, 0)),
            pl.BlockSpec((EB, H), lambda i: (i, 0)),
        ],
        out_shape=[
            jax.ShapeDtypeStruct((E, H), jnp.float32),
            jax.ShapeDtypeStruct((E, H), jnp.float32),
        ],
    )(edge_feats, Wcat, bcat)


NB = 2000


def _mid_a_body(ss_ref, sd_ref, cs_ref, cd_ref, nf_ref, wg1_ref,
                aei_ref, dinv_ref, xw1_ref, y1_ref):
    cs = cs_ref[0, :, :1] + cs_ref[1, :, :1]
    cd = cd_ref[0, :, :1] + cd_ref[1, :, :1]
    raw_row = (ss_ref[0] + ss_ref[1]) / jnp.maximum(cs, 1.0)
    raw_col = (sd_ref[0] + sd_ref[1]) / jnp.maximum(cd, 1.0)
    aei_ref[...] = _sigmoid(raw_row * raw_col)
    dinv = lax.rsqrt(cd + 1.0)
    dinv_ref[...] = jnp.broadcast_to(dinv, dinv_ref.shape)
    xw1 = jnp.dot(nf_ref[...], wg1_ref[...], preferred_element_type=jnp.float32)
    xw1_ref[...] = xw1
    y1_ref[...] = xw1 * dinv


def _mid_a(sumS, sumD, cS, cD, node_feats, W_g1):
    grid = (NPAD // NB,)
    return pl.pallas_call(
        _mid_a_body,
        grid=grid,
        in_specs=[
            pl.BlockSpec((2, NB, H), lambda i: (0, i, 0)),
            pl.BlockSpec((2, NB, H), lambda i: (0, i, 0)),
            pl.BlockSpec((2, NB, 8), lambda i: (0, i, 0)),
            pl.BlockSpec((2, NB, 8), lambda i: (0, i, 0)),
            pl.BlockSpec((NB, D), lambda i: (i, 0)),
            pl.BlockSpec((D, H), lambda i: (0, 0)),
        ],
        out_specs=[
            pl.BlockSpec((NB, H), lambda i: (i, 0)),
            pl.BlockSpec((NB, 8), lambda i: (i, 0)),
            pl.BlockSpec((NB, H), lambda i: (i, 0)),
            pl.BlockSpec((NB, H), lambda i: (i, 0)),
        ],
        out_shape=[
            jax.ShapeDtypeStruct((NPAD, H), jnp.float32),
            jax.ShapeDtypeStruct((NPAD, 8), jnp.float32),
            jax.ShapeDtypeStruct((NPAD, H), jnp.float32),
            jax.ShapeDtypeStruct((NPAD, H), jnp.float32),
        ],
    )(sumS, sumD, cS, cD, node_feats, W_g1)


def _mid_b_body(agg_ref, xw1_ref, dinv_ref, aei_ref, wg2_ref, bg1_ref,
                xw2_ref, y2a_ref, y2b_ref):
    dinv = dinv_ref[:, :1]
    h1 = dinv * (agg_ref[0] + agg_ref[1]) + dinv * dinv * xw1_ref[...] + bg1_ref[...]
    x1 = jnp.maximum(h1, 0.0) * aei_ref[...]
    xw2 = jnp.dot(x1, wg2_ref[...], preferred_element_type=jnp.float32)
    xw2_ref[...] = xw2
    y2 = xw2 * dinv
    y2a_ref[...] = y2[:, :H]
    y2b_ref[...] = y2[:, H:]


def _mid_b(agg1, xw1, dinv8, aei, W_g2, b_g1):
    grid = (NPAD // NB,)
    return pl.pallas_call(
        _mid_b_body,
        grid=grid,
        in_specs=[
            pl.BlockSpec((2, NB, H), lambda i: (0, i, 0)),
            pl.BlockSpec((NB, H), lambda i: (i, 0)),
            pl.BlockSpec((NB, 8), lambda i: (i, 0)),
            pl.BlockSpec((NB, H), lambda i: (i, 0)),
            pl.BlockSpec((H, D), lambda i: (0, 0)),
            pl.BlockSpec((1, H), lambda i: (0, 0)),
        ],
        out_specs=[
            pl.BlockSpec((NB, D), lambda i: (i, 0)),
            pl.BlockSpec((NB, H), lambda i: (i, 0)),
            pl.BlockSpec((NB, H), lambda i: (i, 0)),
        ],
        out_shape=[
            jax.ShapeDtypeStruct((NPAD, D), jnp.float32),
            jax.ShapeDtypeStruct((NPAD, H), jnp.float32),
            jax.ShapeDtypeStruct((NPAD, H), jnp.float32),
        ],
    )(agg1, xw1, dinv8, aei, W_g2, b_g1)


def _mid_c_body(agga_ref, aggb_ref, xw2_ref, dinv_ref, wna_ref, bna_ref,
                wnrt_ref, wnrb_ref, bnr_ref, bg2_ref,
                x_ref, a_ref, b_ref):
    dinv = dinv_ref[:, :1]
    aggtot = jnp.concatenate(
        [agga_ref[0] + agga_ref[1], aggb_ref[0] + aggb_ref[1]], axis=-1)
    h2 = dinv * aggtot + dinv * dinv * xw2_ref[...] + bg2_ref[...]
    x = jnp.maximum(h2, 0.0)
    x_ref[...] = x
    ni = jnp.maximum(
        jnp.dot(x, wna_ref[...], preferred_element_type=jnp.float32) + bna_ref[...],
        0.0)
    a_ref[...] = jnp.dot(ni, wnrt_ref[...], preferred_element_type=jnp.float32) + bnr_ref[...]
    b_ref[...] = jnp.dot(ni, wnrb_ref[...], preferred_element_type=jnp.float32)


def _mid_c(agg2a, agg2b, xw2, dinv8, W_na, b_na, W_nr_t, W_nr_b, b_nr, b_g2):
    grid = (NPAD // NB,)
    return pl.pallas_call(
        _mid_c_body,
        grid=grid,
        in_specs=[
            pl.BlockSpec((2, NB, H), lambda i: (0, i, 0)),
            pl.BlockSpec((2, NB, H), lambda i: (0, i, 0)),
            pl.BlockSpec((NB, D), lambda i: (i, 0)),
            pl.BlockSpec((NB, 8), lambda i: (i, 0)),
            pl.BlockSpec((D, H), lambda i: (0, 0)),
            pl.BlockSpec((1, H), lambda i: (0, 0)),
            pl.BlockSpec((H, H), lambda i: (0, 0)),
            pl.BlockSpec((H, H), lambda i: (0, 0)),
            pl.BlockSpec((1, H), lambda i: (0, 0)),
            pl.BlockSpec((1, D), lambda i: (0, 0)),
        ],
        out_specs=[
            pl.BlockSpec((NB, D), lambda i: (i, 0)),
            pl.BlockSpec((NB, H), lambda i: (i, 0)),
            pl.BlockSpec((NB, H), lambda i: (i, 0)),
        ],
        out_shape=[
            jax.ShapeDtypeStruct((NPAD, D), jnp.float32),
            jax.ShapeDtypeStruct((NPAD, H), jnp.float32),
            jax.ShapeDtypeStruct((NPAD, H), jnp.float32),
        ],
    )(agg2a, agg2b, xw2, dinv8, W_na, b_na, W_nr_t, W_nr_b, b_nr, b_g2)


def _edge_back_body(m1_ref, ra_ref, rb_ref, wm2_ref, bm2_ref, e_ref):
    g = m1_ref[...] * _sigmoid(ra_ref[...] + rb_ref[...])
    e = jnp.dot(g, wm2_ref[...], preferred_element_type=jnp.float32) + bm2_ref[...]
    e_ref[...] = jnp.maximum(e, 0.0)


def _edge_back(m1, rA, rB, W_m2, b_m2):
    grid = (E // EB,)
    return pl.pallas_call(
        _edge_back_body,
        grid=grid,
        in_specs=[
            pl.BlockSpec((EB, H), lambda i: (i, 0)),
            pl.BlockSpec((EB, H), lambda i: (i, 0)),
            pl.BlockSpec((EB, H), lambda i: (i, 0)),
            pl.BlockSpec((H, D), lambda i: (0, 0)),
            pl.BlockSpec((1, D), lambda i: (0, 0)),
        ],
        out_specs=pl.BlockSpec((EB, D), lambda i: (i, 0)),
        out_shape=jax.ShapeDtypeStruct((E, D), jnp.float32),
    )(m1, rA, rB, W_m2, b_m2)


def kernel(node_feats, edge_feats, edge_index, W_g1, b_g1, W_g2, b_g2,
           W_ea, b_ea, W_na, b_na, W_nr, b_nr, W_m1, b_m1, W_m2, b_m2):
    src2 = edge_index[0].reshape(NUNITS, U)
    dst2 = edge_index[1].reshape(NUNITS, U)
    nf_pad = jnp.pad(node_feats, ((0, NPAD - N), (0, 0)))

    Wcat = jnp.concatenate([W_ea, W_m1], axis=1)
    bcat = jnp.concatenate([b_ea, b_m1])[None, :]
    edge_ind, m1 = _edge_front(edge_feats, Wcat, bcat)

    sumS, sumD, cS, cD = _sc_scatter_counts(src2, dst2, edge_ind)
    aei, dinv8, xw1, y1 = _mid_a(sumS, sumD, cS, cD, nf_pad, W_g1)

    agg1 = _sc_gcn_agg(src2, dst2, y1, H)
    xw2, y2a, y2b = _mid_b(agg1, xw1, dinv8, aei, W_g2, b_g1[None, :])

    agg2a = _sc_gcn_agg(src2, dst2, y2a, H)
    agg2b = _sc_gcn_agg(src2, dst2, y2b, H)
    x, A, B = _mid_c(agg2a, agg2b, xw2, dinv8, W_na, b_na[None, :],
                     W_nr[:H], W_nr[H:], b_nr[None, :], b_g2[None, :])

    rA, rB = _sc_edge_gather(src2, dst2, A, B)
    e = _edge_back(m1, rA, rB, W_m2, b_m2[None, :])
    return (x[:N], e)


# speedup vs baseline: 9.4708x; 1.0706x over previous
import functools

import jax
import jax.numpy as jnp
from jax import lax
from jax.experimental import pallas as pl
from jax.experimental.pallas import tpu as pltpu
from jax.experimental.pallas import tpu_sc as plsc

N = 10000
E = 320000
D = 128
H = 64

EB = 8000

NC, NS = 2, 16
NW = NC * NS
U = 80
UPC = 5
CE = U * UPC
NUNITS = E // U
UPW = NUNITS // NW
CPW = UPW // UPC
NPAD = N
RPS = NPAD // NS

_SC_PARAMS = pltpu.CompilerParams(use_tc_tiling_on_sc=False)

_SC_MESH = plsc.VectorSubcoreMesh(
    core_axis_name="c", subcore_axis_name="s", num_cores=NC, num_subcores=NS)


def _sigmoid(x):
    return 1.0 / (1.0 + jnp.exp(-x))


def _worker_coords():
    cid = lax.axis_index("c")
    sid = lax.axis_index("s")
    return cid, sid, sid * NC + cid


def _sc_scatter_counts_body(src2_hbm, dst2_hbm, ei_hbm, z64_hbm, z8_hbm, o8_hbm,
                            outS_hbm, outD_hbm, outCS_hbm, outCD_hbm,
                            sidx2, didx2, vals_v, ones_v,
                            accS, accD, cS8, cD8, sem):
    cid, sid, w = _worker_coords()
    lo = sid * RPS
    pltpu.sync_copy(z64_hbm, accS.at[pl.ds(lo, RPS)])
    pltpu.sync_copy(z64_hbm, accD.at[pl.ds(lo, RPS)])
    pltpu.sync_copy(z8_hbm, cS8.at[pl.ds(lo, RPS)])
    pltpu.sync_copy(z8_hbm, cD8.at[pl.ds(lo, RPS)])
    pltpu.sync_copy(o8_hbm, ones_v)
    plsc.subcore_barrier()

    def chunk(c, carry):
        ub = w * UPW + c * UPC
        pltpu.sync_copy(src2_hbm.at[pl.ds(ub, UPC)], sidx2)
        pltpu.sync_copy(dst2_hbm.at[pl.ds(ub, UPC)], didx2)
        pltpu.sync_copy(ei_hbm.at[pl.ds(ub * U, CE)], vals_v)
        cps = []
        for j in range(UPC):
            v = vals_v.at[pl.ds(j * U, U)]
            cps.append(pltpu.async_copy(v, accS.at[sidx2.at[j]], sem, add=True))
            cps.append(pltpu.async_copy(v, accD.at[didx2.at[j]], sem, add=True))
            cps.append(pltpu.async_copy(ones_v, cS8.at[sidx2.at[j]], sem, add=True))
            cps.append(pltpu.async_copy(ones_v, cD8.at[didx2.at[j]], sem, add=True))
        for cp in cps:
            cp.wait()
        return carry

    lax.fori_loop(0, CPW, chunk, 0)
    plsc.subcore_barrier()
    pltpu.sync_copy(accS.at[pl.ds(lo, RPS)], outS_hbm.at[cid, pl.ds(lo, RPS)])
    pltpu.sync_copy(accD.at[pl.ds(lo, RPS)], outD_hbm.at[cid, pl.ds(lo, RPS)])
    pltpu.sync_copy(cS8.at[pl.ds(lo, RPS)], outCS_hbm.at[cid, pl.ds(lo, RPS)])
    pltpu.sync_copy(cD8.at[pl.ds(lo, RPS)], outCD_hbm.at[cid, pl.ds(lo, RPS)])


def _sc_scatter_counts(src2, dst2, edge_ind):
    z64 = jnp.zeros((RPS, H), jnp.float32)
    z8 = jnp.zeros((RPS, 8), jnp.float32)
    o8 = jnp.ones((U, 8), jnp.float32)
    fn = pl.kernel(
        _sc_scatter_counts_body,
        out_type=[
            jax.ShapeDtypeStruct((NC, NPAD, H), jnp.float32),
            jax.ShapeDtypeStruct((NC, NPAD, H), jnp.float32),
            jax.ShapeDtypeStruct((NC, NPAD, 8), jnp.float32),
            jax.ShapeDtypeStruct((NC, NPAD, 8), jnp.float32),
        ],
        mesh=_SC_MESH,
        compiler_params=_SC_PARAMS,
        scratch_types=[
            pltpu.VMEM((UPC, U), jnp.int32),
            pltpu.VMEM((UPC, U), jnp.int32),
            pltpu.VMEM((CE, H), jnp.float32),
            pltpu.VMEM((U, 8), jnp.float32),
            pltpu.VMEM_SHARED((NPAD, H), jnp.float32),
            pltpu.VMEM_SHARED((NPAD, H), jnp.float32),
            pltpu.VMEM_SHARED((NPAD, 8), jnp.float32),
            pltpu.VMEM_SHARED((NPAD, 8), jnp.float32),
            pltpu.SemaphoreType.DMA,
        ],
    )
    return fn(src2, dst2, edge_ind, z64, z8, o8)


def _sc_gcn_agg_body(src2_hbm, dst2_hbm, y_hbm, zF_hbm, out_hbm,
                     sidx, didx, vals0, vals1, acc, semg, sems0, sems1):
    cid, sid, w = _worker_coords()
    lo = sid * RPS
    pltpu.sync_copy(zF_hbm, acc.at[pl.ds(lo, RPS)])
    pltpu.sync_copy(src2_hbm.at[pl.ds(w * UPW, UPW)], sidx)
    pltpu.sync_copy(dst2_hbm.at[pl.ds(w * UPW, UPW)], didx)
    plsc.subcore_barrier()

    bufs = (vals0, vals1)
    sems = (sems0, sems1)
    pend = [[], []]
    for c in range(CPW):
        b = c % 2
        buf = bufs[b]
        for cp in pend[b]:
            cp.wait()
        pend[b] = []
        gcps = [pltpu.async_copy(y_hbm.at[sidx.at[c * UPC + j]],
                                 buf.at[pl.ds(j * U, U)], semg)
                for j in range(UPC)]
        for j in range(UPC):
            gcps[j].wait()
            pend[b].append(pltpu.async_copy(buf.at[pl.ds(j * U, U)],
                                            acc.at[didx.at[c * UPC + j]],
                                            sems[b], add=True))
    for lst in pend:
        for cp in lst:
            cp.wait()
    plsc.subcore_barrier()
    pltpu.sync_copy(acc.at[pl.ds(lo, RPS)], out_hbm.at[cid, pl.ds(lo, RPS)])


def _sc_gcn_agg(src2, dst2, y, F):
    zF = jnp.zeros((RPS, F), jnp.float32)
    fn = pl.kernel(
        _sc_gcn_agg_body,
        out_type=jax.ShapeDtypeStruct((NC, NPAD, F), jnp.float32),
        mesh=_SC_MESH,
        compiler_params=_SC_PARAMS,
        scratch_types=[
            pltpu.VMEM((UPW, U), jnp.int32),
            pltpu.VMEM((UPW, U), jnp.int32),
            pltpu.VMEM((CE, F), jnp.float32),
            pltpu.VMEM((CE, F), jnp.float32),
            pltpu.VMEM_SHARED((NPAD, F), jnp.float32),
            pltpu.SemaphoreType.DMA,
            pltpu.SemaphoreType.DMA,
            pltpu.SemaphoreType.DMA,
        ],
    )
    return fn(src2, dst2, y, zF)


def _sc_edge_gather_body(src2_hbm, dst2_hbm, a_hbm, b_hbm,
                         outA_hbm, outB_hbm,
                         sidx, didx, vA0, vA1, vB0, vB1, semg, semst0, semst1):
    cid, sid, w = _worker_coords()
    pltpu.sync_copy(src2_hbm.at[pl.ds(w * UPW, UPW)], sidx)
    pltpu.sync_copy(dst2_hbm.at[pl.ds(w * UPW, UPW)], didx)

    bufsA = (vA0, vA1)
    bufsB = (vB0, vB1)
    semst = (semst0, semst1)
    pend = [[], []]
    for c in range(CPW):
        b = c % 2
        ub = w * UPW + c * UPC
        for cp in pend[b]:
            cp.wait()
        pend[b] = []
        gcps = [pltpu.async_copy(a_hbm.at[sidx.at[c * UPC + j]],
                                 bufsA[b].at[pl.ds(j * U, U)], semg)
                for j in range(UPC)]
        gcps += [pltpu.async_copy(b_hbm.at[didx.at[c * UPC + j]],
                                  bufsB[b].at[pl.ds(j * U, U)], semg)
                 for j in range(UPC)]
        for cp in gcps:
            cp.wait()
        pend[b].append(pltpu.async_copy(
            bufsA[b], outA_hbm.at[pl.ds(ub * U, CE)], semst[b]))
        pend[b].append(pltpu.async_copy(
            bufsB[b], outB_hbm.at[pl.ds(ub * U, CE)], semst[b]))
    for lst in pend:
        for cp in lst:
            cp.wait()


def _sc_edge_gather(src2, dst2, A, B):
    fn = pl.kernel(
        _sc_edge_gather_body,
        out_type=[
            jax.ShapeDtypeStruct((E, H), jnp.float32),
            jax.ShapeDtypeStruct((E, H), jnp.float32),
        ],
        mesh=_SC_MESH,
        compiler_params=_SC_PARAMS,
        scratch_types=[
            pltpu.VMEM((UPW, U), jnp.int32),
            pltpu.VMEM((UPW, U), jnp.int32),
            pltpu.VMEM((CE, H), jnp.float32),
            pltpu.VMEM((CE, H), jnp.float32),
            pltpu.VMEM((CE, H), jnp.float32),
            pltpu.VMEM((CE, H), jnp.float32),
            pltpu.SemaphoreType.DMA,
            pltpu.SemaphoreType.DMA,
            pltpu.SemaphoreType.DMA,
        ],
    )
    return fn(src2, dst2, A, B)


def _edge_front_body(ef_ref, w_ref, b_ref, ei_ref, m1_ref):
    p = jnp.dot(ef_ref[...], w_ref[...], preferred_element_type=jnp.float32)
    p = p + b_ref[...]
    ei_ref[...] = p[:, :H]
    m1_ref[...] = jnp.maximum(p[:, H:], 0.0)


def _edge_front(edge_feats, Wcat, bcat):
    grid = (E // EB,)
    return pl.pallas_call(
        _edge_front_body,
        grid=grid,
        in_specs=[
            pl.BlockSpec((EB, D), lambda i: (i, 0)),
            pl.BlockSpec((D, D), lambda i: (0, 0)),
            pl.BlockSpec((1, D), lambda i: (0, 0)),
        ],
        out_specs=[
            pl.BlockSpec((EB, H), lambda i: (i, 0)),
            pl.BlockSpec((EB, H), lambda i: (i, 0)),
        ],
        out_shape=[
            jax.ShapeDtypeStruct((E, H), jnp.float32),
            jax.ShapeDtypeStruct((E, H), jnp.float32),
        ],
    )(edge_feats, Wcat, bcat)


NB = 2000


def _mid_a_body(ss_ref, sd_ref, cs_ref, cd_ref, nf_ref, wg1_ref,
                aei_ref, dinv_ref, xw1_ref, y1_ref):
    cs = cs_ref[0, :, :1] + cs_ref[1, :, :1]
    cd = cd_ref[0, :, :1] + cd_ref[1, :, :1]
    raw_row = (ss_ref[0] + ss_ref[1]) / jnp.maximum(cs, 1.0)
    raw_col = (sd_ref[0] + sd_ref[1]) / jnp.maximum(cd, 1.0)
    aei_ref[...] = _sigmoid(raw_row * raw_col)
    dinv = lax.rsqrt(cd + 1.0)
    dinv_ref[...] = jnp.broadcast_to(dinv, dinv_ref.shape)
    xw1 = jnp.dot(nf_ref[...], wg1_ref[...], preferred_element_type=jnp.float32)
    xw1_ref[...] = xw1
    y1_ref[...] = xw1 * dinv


def _mid_a(sumS, sumD, cS, cD, node_feats, W_g1):
    grid = (NPAD // NB,)
    return pl.pallas_call(
        _mid_a_body,
        grid=grid,
        in_specs=[
            pl.BlockSpec((2, NB, H), lambda i: (0, i, 0)),
            pl.BlockSpec((2, NB, H), lambda i: (0, i, 0)),
            pl.BlockSpec((2, NB, 8), lambda i: (0, i, 0)),
            pl.BlockSpec((2, NB, 8), lambda i: (0, i, 0)),
            pl.BlockSpec((NB, D), lambda i: (i, 0)),
            pl.BlockSpec((D, H), lambda i: (0, 0)),
        ],
        out_specs=[
            pl.BlockSpec((NB, H), lambda i: (i, 0)),
            pl.BlockSpec((NB, 8), lambda i: (i, 0)),
            pl.BlockSpec((NB, H), lambda i: (i, 0)),
            pl.BlockSpec((NB, H), lambda i: (i, 0)),
        ],
        out_shape=[
            jax.ShapeDtypeStruct((NPAD, H), jnp.float32),
            jax.ShapeDtypeStruct((NPAD, 8), jnp.float32),
            jax.ShapeDtypeStruct((NPAD, H), jnp.float32),
            jax.ShapeDtypeStruct((NPAD, H), jnp.float32),
        ],
    )(sumS, sumD, cS, cD, node_feats, W_g1)


def _mid_b_body(agg_ref, xw1_ref, dinv_ref, aei_ref, wg2_ref, bg1_ref,
                xw2_ref, y2a_ref, y2b_ref):
    dinv = dinv_ref[:, :1]
    h1 = dinv * (agg_ref[0] + agg_ref[1]) + dinv * dinv * xw1_ref[...] + bg1_ref[...]
    x1 = jnp.maximum(h1, 0.0) * aei_ref[...]
    xw2 = jnp.dot(x1, wg2_ref[...], preferred_element_type=jnp.float32)
    xw2_ref[...] = xw2
    y2 = xw2 * dinv
    y2a_ref[...] = y2[:, :H]
    y2b_ref[...] = y2[:, H:]


def _mid_b(agg1, xw1, dinv8, aei, W_g2, b_g1):
    grid = (NPAD // NB,)
    return pl.pallas_call(
        _mid_b_body,
        grid=grid,
        in_specs=[
            pl.BlockSpec((2, NB, H), lambda i: (0, i, 0)),
            pl.BlockSpec((NB, H), lambda i: (i, 0)),
            pl.BlockSpec((NB, 8), lambda i: (i, 0)),
            pl.BlockSpec((NB, H), lambda i: (i, 0)),
            pl.BlockSpec((H, D), lambda i: (0, 0)),
            pl.BlockSpec((1, H), lambda i: (0, 0)),
        ],
        out_specs=[
            pl.BlockSpec((NB, D), lambda i: (i, 0)),
            pl.BlockSpec((NB, H), lambda i: (i, 0)),
            pl.BlockSpec((NB, H), lambda i: (i, 0)),
        ],
        out_shape=[
            jax.ShapeDtypeStruct((NPAD, D), jnp.float32),
            jax.ShapeDtypeStruct((NPAD, H), jnp.float32),
            jax.ShapeDtypeStruct((NPAD, H), jnp.float32),
        ],
    )(agg1, xw1, dinv8, aei, W_g2, b_g1)


def _mid_c_body(agga_ref, aggb_ref, xw2_ref, dinv_ref, wna_ref, bna_ref,
                wnrt_ref, wnrb_ref, bnr_ref, bg2_ref,
                x_ref, a_ref, b_ref):
    dinv = dinv_ref[:, :1]
    aggtot = jnp.concatenate(
        [agga_ref[0] + agga_ref[1], aggb_ref[0] + aggb_ref[1]], axis=-1)
    h2 = dinv * aggtot + dinv * dinv * xw2_ref[...] + bg2_ref[...]
    x = jnp.maximum(h2, 0.0)
    x_ref[...] = x
    ni = jnp.maximum(
        jnp.dot(x, wna_ref[...], preferred_element_type=jnp.float32) + bna_ref[...],
        0.0)
    a_ref[...] = jnp.dot(ni, wnrt_ref[...], preferred_element_type=jnp.float32) + bnr_ref[...]
    b_ref[...] = jnp.dot(ni, wnrb_ref[...], preferred_element_type=jnp.float32)


def _mid_c(agg2a, agg2b, xw2, dinv8, W_na, b_na, W_nr_t, W_nr_b, b_nr, b_g2):
    grid = (NPAD // NB,)
    return pl.pallas_call(
        _mid_c_body,
        grid=grid,
        in_specs=[
            pl.BlockSpec((2, NB, H), lambda i: (0, i, 0)),
            pl.BlockSpec((2, NB, H), lambda i: (0, i, 0)),
            pl.BlockSpec((NB, D), lambda i: (i, 0)),
            pl.BlockSpec((NB, 8), lambda i: (i, 0)),
            pl.BlockSpec((D, H), lambda i: (0, 0)),
            pl.BlockSpec((1, H), lambda i: (0, 0)),
            pl.BlockSpec((H, H), lambda i: (0, 0)),
            pl.BlockSpec((H, H), lambda i: (0, 0)),
            pl.BlockSpec((1, H), lambda i: (0, 0)),
            pl.BlockSpec((1, D), lambda i: (0, 0)),
        ],
        out_specs=[
            pl.BlockSpec((NB, D), lambda i: (i, 0)),
            pl.BlockSpec((NB, H), lambda i: (i, 0)),
            pl.BlockSpec((NB, H), lambda i: (i, 0)),
        ],
        out_shape=[
            jax.ShapeDtypeStruct((NPAD, D), jnp.float32),
            jax.ShapeDtypeStruct((NPAD, H), jnp.float32),
            jax.ShapeDtypeStruct((NPAD, H), jnp.float32),
        ],
    )(agg2a, agg2b, xw2, dinv8, W_na, b_na, W_nr_t, W_nr_b, b_nr, b_g2)


def _edge_back_body(m1_ref, ra_ref, rb_ref, wm2_ref, bm2_ref, e_ref):
    g = m1_ref[...] * _sigmoid(ra_ref[...] + rb_ref[...])
    e = jnp.dot(g, wm2_ref[...], preferred_element_type=jnp.float32) + bm2_ref[...]
    e_ref[...] = jnp.maximum(e, 0.0)


def _edge_back(m1, rA, rB, W_m2, b_m2):
    grid = (E // EB,)
    return pl.pallas_call(
        _edge_back_body,
        grid=grid,
        in_specs=[
            pl.BlockSpec((EB, H), lambda i: (i, 0)),
            pl.BlockSpec((EB, H), lambda i: (i, 0)),
            pl.BlockSpec((EB, H), lambda i: (i, 0)),
            pl.BlockSpec((H, D), lambda i: (0, 0)),
            pl.BlockSpec((1, D), lambda i: (0, 0)),
        ],
        out_specs=pl.BlockSpec((EB, D), lambda i: (i, 0)),
        out_shape=jax.ShapeDtypeStruct((E, D), jnp.float32),
    )(m1, rA, rB, W_m2, b_m2)


def kernel(node_feats, edge_feats, edge_index, W_g1, b_g1, W_g2, b_g2,
           W_ea, b_ea, W_na, b_na, W_nr, b_nr, W_m1, b_m1, W_m2, b_m2):
    src2 = edge_index[0].reshape(NUNITS, U)
    dst2 = edge_index[1].reshape(NUNITS, U)
    nf_pad = jnp.pad(node_feats, ((0, NPAD - N), (0, 0)))

    Wcat = jnp.concatenate([W_ea, W_m1], axis=1)
    bcat = jnp.concatenate([b_ea, b_m1])[None, :]
    edge_ind, m1 = _edge_front(edge_feats, Wcat, bcat)

    sumS, sumD, cS, cD = _sc_scatter_counts(src2, dst2, edge_ind)
    aei, dinv8, xw1, y1 = _mid_a(sumS, sumD, cS, cD, nf_pad, W_g1)

    agg1 = _sc_gcn_agg(src2, dst2, y1, H)
    xw2, y2a, y2b = _mid_b(agg1, xw1, dinv8, aei, W_g2, b_g1[None, :])

    agg2a = _sc_gcn_agg(src2, dst2, y2a, H)
    agg2b = _sc_gcn_agg(src2, dst2, y2b, H)
    x, A, B = _mid_c(agg2a, agg2b, xw2, dinv8, W_na, b_na[None, :],
                     W_nr[:H], W_nr[H:], b_nr[None, :], b_g2[None, :])

    rA, rB = _sc_edge_gather(src2, dst2, A, B)
    e = _edge_back(m1, rA, rB, W_m2, b_m2[None, :])
    return (x[:N], e)


# speedup vs baseline: 11.1175x; 1.1739x over previous
import functools

import jax
import jax.numpy as jnp
from jax import lax
from jax.experimental import pallas as pl
from jax.experimental.pallas import tpu as pltpu
from jax.experimental.pallas import tpu_sc as plsc

N = 10000
E = 320000
D = 128
H = 64

EB = 8000

NC, NS = 2, 16
NW = NC * NS
U = 80
UPC = 5
CE = U * UPC
NUNITS = E // U
UPW = NUNITS // NW
CPW = UPW // UPC
NPAD = N
RPS = NPAD // NS

_SC_PARAMS = pltpu.CompilerParams(use_tc_tiling_on_sc=False)

_SC_MESH = plsc.VectorSubcoreMesh(
    core_axis_name="c", subcore_axis_name="s", num_cores=NC, num_subcores=NS)


def _sigmoid(x):
    return 1.0 / (1.0 + jnp.exp(-x))


def _worker_coords():
    cid = lax.axis_index("c")
    sid = lax.axis_index("s")
    return cid, sid, sid * NC + cid


def _sc_scatter_counts_body(src2_hbm, dst2_hbm, ei_hbm, z64_hbm, z8_hbm, o8_hbm,
                            outS_hbm, outD_hbm, outCS_hbm, outCD_hbm,
                            sidx2, didx2, vals_v, ones_v,
                            accS, accD, cS8, cD8, sem):
    cid, sid, w = _worker_coords()
    lo = sid * RPS
    pltpu.sync_copy(z64_hbm, accS.at[pl.ds(lo, RPS)])
    pltpu.sync_copy(z64_hbm, accD.at[pl.ds(lo, RPS)])
    pltpu.sync_copy(z8_hbm, cS8.at[pl.ds(lo, RPS)])
    pltpu.sync_copy(z8_hbm, cD8.at[pl.ds(lo, RPS)])
    pltpu.sync_copy(o8_hbm, ones_v)
    plsc.subcore_barrier()

    def chunk(c, carry):
        ub = w * UPW + c * UPC
        pltpu.sync_copy(src2_hbm.at[pl.ds(ub, UPC)], sidx2)
        pltpu.sync_copy(dst2_hbm.at[pl.ds(ub, UPC)], didx2)
        pltpu.sync_copy(ei_hbm.at[pl.ds(ub * U, CE)], vals_v)
        cps = []
        for j in range(UPC):
            v = vals_v.at[pl.ds(j * U, U)]
            cps.append(pltpu.async_copy(v, accS.at[sidx2.at[j]], sem, add=True))
            cps.append(pltpu.async_copy(v, accD.at[didx2.at[j]], sem, add=True))
            cps.append(pltpu.async_copy(ones_v, cS8.at[sidx2.at[j]], sem, add=True))
            cps.append(pltpu.async_copy(ones_v, cD8.at[didx2.at[j]], sem, add=True))
        for cp in cps:
            cp.wait()
        return carry

    lax.fori_loop(0, CPW, chunk, 0)
    plsc.subcore_barrier()
    pltpu.sync_copy(accS.at[pl.ds(lo, RPS)], outS_hbm.at[cid, pl.ds(lo, RPS)])
    pltpu.sync_copy(accD.at[pl.ds(lo, RPS)], outD_hbm.at[cid, pl.ds(lo, RPS)])
    pltpu.sync_copy(cS8.at[pl.ds(lo, RPS)], outCS_hbm.at[cid, pl.ds(lo, RPS)])
    pltpu.sync_copy(cD8.at[pl.ds(lo, RPS)], outCD_hbm.at[cid, pl.ds(lo, RPS)])


def _sc_scatter_counts(src2, dst2, edge_ind):
    z64 = jnp.zeros((RPS, H), jnp.float32)
    z8 = jnp.zeros((RPS, 8), jnp.float32)
    o8 = jnp.ones((U, 8), jnp.float32)
    fn = pl.kernel(
        _sc_scatter_counts_body,
        out_type=[
            jax.ShapeDtypeStruct((NC, NPAD, H), jnp.float32),
            jax.ShapeDtypeStruct((NC, NPAD, H), jnp.float32),
            jax.ShapeDtypeStruct((NC, NPAD, 8), jnp.float32),
            jax.ShapeDtypeStruct((NC, NPAD, 8), jnp.float32),
        ],
        mesh=_SC_MESH,
        compiler_params=_SC_PARAMS,
        scratch_types=[
            pltpu.VMEM((UPC, U), jnp.int32),
            pltpu.VMEM((UPC, U), jnp.int32),
            pltpu.VMEM((CE, H), jnp.float32),
            pltpu.VMEM((U, 8), jnp.float32),
            pltpu.VMEM_SHARED((NPAD, H), jnp.float32),
            pltpu.VMEM_SHARED((NPAD, H), jnp.float32),
            pltpu.VMEM_SHARED((NPAD, 8), jnp.float32),
            pltpu.VMEM_SHARED((NPAD, 8), jnp.float32),
            pltpu.SemaphoreType.DMA,
        ],
    )
    return fn(src2, dst2, edge_ind, z64, z8, o8)


def _sc_gcn_agg_body(src2_hbm, dst2_hbm, y_hbm, zF_hbm, out_hbm,
                     sidx, didx, vals0, vals1, acc, semg, sems0, sems1):
    cid, sid, w = _worker_coords()
    lo = sid * RPS
    pltpu.sync_copy(zF_hbm, acc.at[pl.ds(lo, RPS)])
    pltpu.sync_copy(src2_hbm.at[pl.ds(w * UPW, UPW)], sidx)
    pltpu.sync_copy(dst2_hbm.at[pl.ds(w * UPW, UPW)], didx)
    plsc.subcore_barrier()

    bufs = (vals0, vals1)
    sems = (sems0, sems1)
    pend = [[], []]
    for c in range(CPW):
        b = c % 2
        buf = bufs[b]
        for cp in pend[b]:
            cp.wait()
        pend[b] = []
        gcps = [pltpu.async_copy(y_hbm.at[sidx.at[c * UPC + j]],
                                 buf.at[pl.ds(j * U, U)], semg)
                for j in range(UPC)]
        for j in range(UPC):
            gcps[j].wait()
            pend[b].append(pltpu.async_copy(buf.at[pl.ds(j * U, U)],
                                            acc.at[didx.at[c * UPC + j]],
                                            sems[b], add=True))
    for lst in pend:
        for cp in lst:
            cp.wait()
    plsc.subcore_barrier()
    pltpu.sync_copy(acc.at[pl.ds(lo, RPS)], out_hbm.at[cid, pl.ds(lo, RPS)])


def _sc_gcn_agg(src2, dst2, y, F):
    zF = jnp.zeros((RPS, F), jnp.float32)
    fn = pl.kernel(
        _sc_gcn_agg_body,
        out_type=jax.ShapeDtypeStruct((NC, NPAD, F), jnp.float32),
        mesh=_SC_MESH,
        compiler_params=_SC_PARAMS,
        scratch_types=[
            pltpu.VMEM((UPW, U), jnp.int32),
            pltpu.VMEM((UPW, U), jnp.int32),
            pltpu.VMEM((CE, F), jnp.float32),
            pltpu.VMEM((CE, F), jnp.float32),
            pltpu.VMEM_SHARED((NPAD, F), jnp.float32),
            pltpu.SemaphoreType.DMA,
            pltpu.SemaphoreType.DMA,
            pltpu.SemaphoreType.DMA,
        ],
    )
    return fn(src2, dst2, y, zF)


UD = 40
UPCD = 10
CED = UD * UPCD
NUNITSD = E // UD
UPWD = NUNITSD // NW
CPWD = UPWD // UPCD


def _sc_edge_gather_body(iv2_hbm, t_hbm, out_hbm,
                         ividx, v0, v1, semg, semst0, semst1):
    cid, sid, w = _worker_coords()
    pltpu.sync_copy(iv2_hbm.at[pl.ds(w * UPWD, UPWD)], ividx)

    bufs = (v0, v1)
    semst = (semst0, semst1)
    pend = [[], []]
    for c in range(CPWD):
        b = c % 2
        ub = w * UPWD + c * UPCD
        for cp in pend[b]:
            cp.wait()
        pend[b] = []
        gcps = [pltpu.async_copy(t_hbm.at[ividx.at[c * UPCD + j]],
                                 bufs[b].at[pl.ds(j * 2 * UD, 2 * UD)], semg)
                for j in range(UPCD)]
        for cp in gcps:
            cp.wait()
        pend[b].append(pltpu.async_copy(
            bufs[b], out_hbm.at[pl.ds(ub * 2 * UD, 2 * CED)], semst[b]))
    for lst in pend:
        for cp in lst:
            cp.wait()


def _sc_edge_gather(iv2, T):
    fn = pl.kernel(
        _sc_edge_gather_body,
        out_type=jax.ShapeDtypeStruct((2 * E, H), jnp.float32),
        mesh=_SC_MESH,
        compiler_params=_SC_PARAMS,
        scratch_types=[
            pltpu.VMEM((UPWD, 2 * UD), jnp.int32),
            pltpu.VMEM((2 * CED, H), jnp.float32),
            pltpu.VMEM((2 * CED, H), jnp.float32),
            pltpu.SemaphoreType.DMA,
            pltpu.SemaphoreType.DMA,
            pltpu.SemaphoreType.DMA,
        ],
    )
    return fn(iv2, T)


def _edge_front_body(ef_ref, w_ref, b_ref, ei_ref, m1_ref):
    p = jnp.dot(ef_ref[...], w_ref[...], preferred_element_type=jnp.float32)
    p = p + b_ref[...]
    ei_ref[...] = p[:, :H]
    m1_ref[...] = jnp.maximum(p[:, H:], 0.0)


def _edge_front(edge_feats, Wcat, bcat):
    grid = (E // EB,)
    return pl.pallas_call(
        _edge_front_body,
        grid=grid,
        in_specs=[
            pl.BlockSpec((EB, D), lambda i: (i, 0)),
            pl.BlockSpec((D, D), lambda i: (0, 0)),
            pl.BlockSpec((1, D), lambda i: (0, 0)),
        ],
        out_specs=[
            pl.BlockSpec((EB, H), lambda i: (i, 0)),
            pl.BlockSpec((EB, H), lambda i: (i, 0)),
        ],
        out_shape=[
            jax.ShapeDtypeStruct((E, H), jnp.float32),
            jax.ShapeDtypeStruct((E, H), jnp.float32),
        ],
    )(edge_feats, Wcat, bcat)


NB = 2000


def _mid_a_body(ss_ref, sd_ref, cs_ref, cd_ref, nf_ref, wg1_ref,
                aei_ref, dinv_ref, xw1_ref, y1_ref):
    cs = cs_ref[0, :, :1] + cs_ref[1, :, :1]
    cd = cd_ref[0, :, :1] + cd_ref[1, :, :1]
    raw_row = (ss_ref[0] + ss_ref[1]) / jnp.maximum(cs, 1.0)
    raw_col = (sd_ref[0] + sd_ref[1]) / jnp.maximum(cd, 1.0)
    aei_ref[...] = _sigmoid(raw_row * raw_col)
    dinv = lax.rsqrt(cd + 1.0)
    dinv_ref[...] = jnp.broadcast_to(dinv, dinv_ref.shape)
    xw1 = jnp.dot(nf_ref[...], wg1_ref[...], preferred_element_type=jnp.float32)
    xw1_ref[...] = xw1
    y1_ref[...] = xw1 * dinv


def _mid_a(sumS, sumD, cS, cD, node_feats, W_g1):
    grid = (NPAD // NB,)
    return pl.pallas_call(
        _mid_a_body,
        grid=grid,
        in_specs=[
            pl.BlockSpec((2, NB, H), lambda i: (0, i, 0)),
            pl.BlockSpec((2, NB, H), lambda i: (0, i, 0)),
            pl.BlockSpec((2, NB, 8), lambda i: (0, i, 0)),
            pl.BlockSpec((2, NB, 8), lambda i: (0, i, 0)),
            pl.BlockSpec((NB, D), lambda i: (i, 0)),
            pl.BlockSpec((D, H), lambda i: (0, 0)),
        ],
        out_specs=[
            pl.BlockSpec((NB, H), lambda i: (i, 0)),
            pl.BlockSpec((NB, 8), lambda i: (i, 0)),
            pl.BlockSpec((NB, H), lambda i: (i, 0)),
            pl.BlockSpec((NB, H), lambda i: (i, 0)),
        ],
        out_shape=[
            jax.ShapeDtypeStruct((NPAD, H), jnp.float32),
            jax.ShapeDtypeStruct((NPAD, 8), jnp.float32),
            jax.ShapeDtypeStruct((NPAD, H), jnp.float32),
            jax.ShapeDtypeStruct((NPAD, H), jnp.float32),
        ],
    )(sumS, sumD, cS, cD, node_feats, W_g1)


def _mid_b_body(agg_ref, xw1_ref, dinv_ref, aei_ref, wg2_ref, bg1_ref,
                xw2_ref, y2a_ref, y2b_ref):
    dinv = dinv_ref[:, :1]
    h1 = dinv * (agg_ref[0] + agg_ref[1]) + dinv * dinv * xw1_ref[...] + bg1_ref[...]
    x1 = jnp.maximum(h1, 0.0) * aei_ref[...]
    xw2 = jnp.dot(x1, wg2_ref[...], preferred_element_type=jnp.float32)
    xw2_ref[...] = xw2
    y2 = xw2 * dinv
    y2a_ref[...] = y2[:, :H]
    y2b_ref[...] = y2[:, H:]


def _mid_b(agg1, xw1, dinv8, aei, W_g2, b_g1):
    grid = (NPAD // NB,)
    return pl.pallas_call(
        _mid_b_body,
        grid=grid,
        in_specs=[
            pl.BlockSpec((2, NB, H), lambda i: (0, i, 0)),
            pl.BlockSpec((NB, H), lambda i: (i, 0)),
            pl.BlockSpec((NB, 8), lambda i: (i, 0)),
            pl.BlockSpec((NB, H), lambda i: (i, 0)),
            pl.BlockSpec((H, D), lambda i: (0, 0)),
            pl.BlockSpec((1, H), lambda i: (0, 0)),
        ],
        out_specs=[
            pl.BlockSpec((NB, D), lambda i: (i, 0)),
            pl.BlockSpec((NB, H), lambda i: (i, 0)),
            pl.BlockSpec((NB, H), lambda i: (i, 0)),
        ],
        out_shape=[
            jax.ShapeDtypeStruct((NPAD, D), jnp.float32),
            jax.ShapeDtypeStruct((NPAD, H), jnp.float32),
            jax.ShapeDtypeStruct((NPAD, H), jnp.float32),
        ],
    )(agg1, xw1, dinv8, aei, W_g2, b_g1)


def _mid_c_body(agga_ref, aggb_ref, xw2_ref, dinv_ref, wna_ref, bna_ref,
                wnrt_ref, wnrb_ref, bnr_ref, bg2_ref,
                x_ref, t_ref):
    dinv = dinv_ref[:, :1]
    aggtot = jnp.concatenate(
        [agga_ref[0] + agga_ref[1], aggb_ref[0] + aggb_ref[1]], axis=-1)
    h2 = dinv * aggtot + dinv * dinv * xw2_ref[...] + bg2_ref[...]
    x = jnp.maximum(h2, 0.0)
    x_ref[...] = x
    ni = jnp.maximum(
        jnp.dot(x, wna_ref[...], preferred_element_type=jnp.float32) + bna_ref[...],
        0.0)
    a = jnp.dot(ni, wnrt_ref[...], preferred_element_type=jnp.float32) + bnr_ref[...]
    b = jnp.dot(ni, wnrb_ref[...], preferred_element_type=jnp.float32)
    t_ref[...] = jnp.concatenate([a, b], axis=-1)


def _mid_c(agg2a, agg2b, xw2, dinv8, W_na, b_na, W_nr_t, W_nr_b, b_nr, b_g2):
    grid = (NPAD // NB,)
    return pl.pallas_call(
        _mid_c_body,
        grid=grid,
        in_specs=[
            pl.BlockSpec((2, NB, H), lambda i: (0, i, 0)),
            pl.BlockSpec((2, NB, H), lambda i: (0, i, 0)),
            pl.BlockSpec((NB, D), lambda i: (i, 0)),
            pl.BlockSpec((NB, 8), lambda i: (i, 0)),
            pl.BlockSpec((D, H), lambda i: (0, 0)),
            pl.BlockSpec((1, H), lambda i: (0, 0)),
            pl.BlockSpec((H, H), lambda i: (0, 0)),
            pl.BlockSpec((H, H), lambda i: (0, 0)),
            pl.BlockSpec((1, H), lambda i: (0, 0)),
            pl.BlockSpec((1, D), lambda i: (0, 0)),
        ],
        out_specs=[
            pl.BlockSpec((NB, D), lambda i: (i, 0)),
            pl.BlockSpec((NB, D), lambda i: (i, 0)),
        ],
        out_shape=[
            jax.ShapeDtypeStruct((NPAD, D), jnp.float32),
            jax.ShapeDtypeStruct((NPAD, D), jnp.float32),
        ],
    )(agg2a, agg2b, xw2, dinv8, W_na, b_na, W_nr_t, W_nr_b, b_nr, b_g2)


def _edge_back_body(m1_ref, g2_ref, wm2_ref, bm2_ref, e_ref):
    g2 = g2_ref[...]
    g = m1_ref[...] * _sigmoid(g2[:, :H] + g2[:, H:])
    e = jnp.dot(g, wm2_ref[...], preferred_element_type=jnp.float32) + bm2_ref[...]
    e_ref[...] = jnp.maximum(e, 0.0)


def _edge_back(m1, g2, W_m2, b_m2):
    grid = (E // EB,)
    return pl.pallas_call(
        _edge_back_body,
        grid=grid,
        in_specs=[
            pl.BlockSpec((EB, H), lambda i: (i, 0)),
            pl.BlockSpec((EB, D), lambda i: (i, 0)),
            pl.BlockSpec((H, D), lambda i: (0, 0)),
            pl.BlockSpec((1, D), lambda i: (0, 0)),
        ],
        out_specs=pl.BlockSpec((EB, D), lambda i: (i, 0)),
        out_shape=jax.ShapeDtypeStruct((E, D), jnp.float32),
    )(m1, g2, W_m2, b_m2)


def kernel(node_feats, edge_feats, edge_index, W_g1, b_g1, W_g2, b_g2,
           W_ea, b_ea, W_na, b_na, W_nr, b_nr, W_m1, b_m1, W_m2, b_m2):
    src2 = edge_index[0].reshape(NUNITS, U)
    dst2 = edge_index[1].reshape(NUNITS, U)
    nf_pad = jnp.pad(node_feats, ((0, NPAD - N), (0, 0)))

    Wcat = jnp.concatenate([W_ea, W_m1], axis=1)
    bcat = jnp.concatenate([b_ea, b_m1])[None, :]
    edge_ind, m1 = _edge_front(edge_feats, Wcat, bcat)

    sumS, sumD, cS, cD = _sc_scatter_counts(src2, dst2, edge_ind)
    aei, dinv8, xw1, y1 = _mid_a(sumS, sumD, cS, cD, nf_pad, W_g1)

    agg1 = _sc_gcn_agg(src2, dst2, y1, H)
    xw2, y2a, y2b = _mid_b(agg1, xw1, dinv8, aei, W_g2, b_g1[None, :])

    agg2a = _sc_gcn_agg(src2, dst2, y2a, H)
    agg2b = _sc_gcn_agg(src2, dst2, y2b, H)
    x, Tcat = _mid_c(agg2a, agg2b, xw2, dinv8, W_na, b_na[None, :],
                     W_nr[:H], W_nr[H:], b_nr[None, :], b_g2[None, :])

    iv2 = jnp.stack([2 * edge_index[0], 2 * edge_index[1] + 1],
                    axis=-1).reshape(NUNITSD, 2 * UD)
    T = Tcat.reshape(2 * NPAD, H)
    g2 = _sc_edge_gather(iv2, T).reshape(E, D)
    e = _edge_back(m1, g2, W_m2, b_m2[None, :])
    return (x[:N], e)


# speedup vs baseline: 11.1191x; 1.0001x over previous
import jax
import jax.numpy as jnp
from jax import lax
from jax.experimental import pallas as pl
from jax.experimental.pallas import tpu as pltpu
from jax.experimental.pallas import tpu_sc as plsc

N = 10000
E = 320000
D = 128
H = 64

EB = 8000

NC, NS = 2, 16
NW = NC * NS
U = 80
UPC = 5
CE = U * UPC
NUNITS = E // U
UPW = NUNITS // NW
CPW = UPW // UPC
NPAD = N
RPS = NPAD // NS

_SC_PARAMS = pltpu.CompilerParams(use_tc_tiling_on_sc=False)

_SC_MESH = plsc.VectorSubcoreMesh(
    core_axis_name="c", subcore_axis_name="s", num_cores=NC, num_subcores=NS)


def _sigmoid(x):
    return 1.0 / (1.0 + jnp.exp(-x))


def _worker_coords():
    cid = lax.axis_index("c")
    sid = lax.axis_index("s")
    return cid, sid, sid * NC + cid


def _sc_scatter_counts_body(src2_hbm, dst2_hbm, ei_hbm, z64_hbm, z8_hbm, o8_hbm,
                            outS_hbm, outD_hbm, outCS_hbm, outCD_hbm,
                            sidx2, didx2, vals_v, ones_v,
                            accS, accD, cS8, cD8, sem):
    cid, sid, w = _worker_coords()
    lo = sid * RPS
    pltpu.sync_copy(z64_hbm, accS.at[pl.ds(lo, RPS)])
    pltpu.sync_copy(z64_hbm, accD.at[pl.ds(lo, RPS)])
    pltpu.sync_copy(z8_hbm, cS8.at[pl.ds(lo, RPS)])
    pltpu.sync_copy(z8_hbm, cD8.at[pl.ds(lo, RPS)])
    pltpu.sync_copy(o8_hbm, ones_v)
    plsc.subcore_barrier()

    def chunk(c, carry):
        ub = w * UPW + c * UPC
        pltpu.sync_copy(src2_hbm.at[pl.ds(ub, UPC)], sidx2)
        pltpu.sync_copy(dst2_hbm.at[pl.ds(ub, UPC)], didx2)
        pltpu.sync_copy(ei_hbm.at[pl.ds(ub * U, CE)], vals_v)
        cps = []
        for j in range(UPC):
            v = vals_v.at[pl.ds(j * U, U)]
            cps.append(pltpu.async_copy(v, accS.at[sidx2.at[j]], sem, add=True))
            cps.append(pltpu.async_copy(v, accD.at[didx2.at[j]], sem, add=True))
            cps.append(pltpu.async_copy(ones_v, cS8.at[sidx2.at[j]], sem, add=True))
            cps.append(pltpu.async_copy(ones_v, cD8.at[didx2.at[j]], sem, add=True))
        for cp in cps:
            cp.wait()
        return carry

    lax.fori_loop(0, CPW, chunk, 0)
    plsc.subcore_barrier()
    pltpu.sync_copy(accS.at[pl.ds(lo, RPS)], outS_hbm.at[cid, pl.ds(lo, RPS)])
    pltpu.sync_copy(accD.at[pl.ds(lo, RPS)], outD_hbm.at[cid, pl.ds(lo, RPS)])
    pltpu.sync_copy(cS8.at[pl.ds(lo, RPS)], outCS_hbm.at[cid, pl.ds(lo, RPS)])
    pltpu.sync_copy(cD8.at[pl.ds(lo, RPS)], outCD_hbm.at[cid, pl.ds(lo, RPS)])


def _sc_scatter_counts(src2, dst2, edge_ind):
    z64 = jnp.zeros((RPS, H), jnp.float32)
    z8 = jnp.zeros((RPS, 8), jnp.float32)
    o8 = jnp.ones((U, 8), jnp.float32)
    fn = pl.kernel(
        _sc_scatter_counts_body,
        out_type=[
            jax.ShapeDtypeStruct((NC, NPAD, H), jnp.float32),
            jax.ShapeDtypeStruct((NC, NPAD, H), jnp.float32),
            jax.ShapeDtypeStruct((NC, NPAD, 8), jnp.float32),
            jax.ShapeDtypeStruct((NC, NPAD, 8), jnp.float32),
        ],
        mesh=_SC_MESH,
        compiler_params=_SC_PARAMS,
        scratch_types=[
            pltpu.VMEM((UPC, U), jnp.int32),
            pltpu.VMEM((UPC, U), jnp.int32),
            pltpu.VMEM((CE, H), jnp.float32),
            pltpu.VMEM((U, 8), jnp.float32),
            pltpu.VMEM_SHARED((NPAD, H), jnp.float32),
            pltpu.VMEM_SHARED((NPAD, H), jnp.float32),
            pltpu.VMEM_SHARED((NPAD, 8), jnp.float32),
            pltpu.VMEM_SHARED((NPAD, 8), jnp.float32),
            pltpu.SemaphoreType.DMA,
        ],
    )
    return fn(src2, dst2, edge_ind, z64, z8, o8)


def _sc_gcn_agg_body(src2_hbm, dst2_hbm, y_hbm, zF_hbm, out_hbm,
                     sidx, didx, vals0, vals1, acc, semg, sems0, sems1):
    cid, sid, w = _worker_coords()
    lo = sid * RPS
    pltpu.sync_copy(zF_hbm, acc.at[pl.ds(lo, RPS)])
    pltpu.sync_copy(src2_hbm.at[pl.ds(w * UPW, UPW)], sidx)
    pltpu.sync_copy(dst2_hbm.at[pl.ds(w * UPW, UPW)], didx)
    plsc.subcore_barrier()

    bufs = (vals0, vals1)
    sems = (sems0, sems1)
    pend = [[], []]
    for c in range(CPW):
        b = c % 2
        buf = bufs[b]
        for cp in pend[b]:
            cp.wait()
        pend[b] = []
        gcps = [pltpu.async_copy(y_hbm.at[sidx.at[c * UPC + j]],
                                 buf.at[pl.ds(j * U, U)], semg)
                for j in range(UPC)]
        for j in range(UPC):
            gcps[j].wait()
            pend[b].append(pltpu.async_copy(buf.at[pl.ds(j * U, U)],
                                            acc.at[didx.at[c * UPC + j]],
                                            sems[b], add=True))
    for lst in pend:
        for cp in lst:
            cp.wait()
    plsc.subcore_barrier()
    pltpu.sync_copy(acc.at[pl.ds(lo, RPS)], out_hbm.at[cid, pl.ds(lo, RPS)])


def _sc_gcn_agg(src2, dst2, y, F):
    zF = jnp.zeros((RPS, F), jnp.float32)
    fn = pl.kernel(
        _sc_gcn_agg_body,
        out_type=jax.ShapeDtypeStruct((NC, NPAD, F), jnp.float32),
        mesh=_SC_MESH,
        compiler_params=_SC_PARAMS,
        scratch_types=[
            pltpu.VMEM((UPW, U), jnp.int32),
            pltpu.VMEM((UPW, U), jnp.int32),
            pltpu.VMEM((CE, F), jnp.float32),
            pltpu.VMEM((CE, F), jnp.float32),
            pltpu.VMEM_SHARED((NPAD, F), jnp.float32),
            pltpu.SemaphoreType.DMA,
            pltpu.SemaphoreType.DMA,
            pltpu.SemaphoreType.DMA,
        ],
    )
    return fn(src2, dst2, y, zF)


UD = 40
UPCD = 10
CED = UD * UPCD
NUNITSD = E // UD
UPWD = NUNITSD // NW
CPWD = UPWD // UPCD


def _sc_edge_gather_body(iv2_hbm, t_hbm, out_hbm,
                         ividx, v0, v1, semg, semst0, semst1):
    cid, sid, w = _worker_coords()
    pltpu.sync_copy(iv2_hbm.at[pl.ds(w * UPWD, UPWD)], ividx)

    bufs = (v0, v1)
    semst = (semst0, semst1)
    pend = [[], []]
    for c in range(CPWD):
        b = c % 2
        ub = w * UPWD + c * UPCD
        for cp in pend[b]:
            cp.wait()
        pend[b] = []
        gcps = [pltpu.async_copy(t_hbm.at[ividx.at[c * UPCD + j]],
                                 bufs[b].at[pl.ds(j * 2 * UD, 2 * UD)], semg)
                for j in range(UPCD)]
        for cp in gcps:
            cp.wait()
        pend[b].append(pltpu.async_copy(
            bufs[b], out_hbm.at[pl.ds(ub * 2 * UD, 2 * CED)], semst[b]))
    for lst in pend:
        for cp in lst:
            cp.wait()


def _sc_edge_gather(iv2, T):
    fn = pl.kernel(
        _sc_edge_gather_body,
        out_type=jax.ShapeDtypeStruct((2 * E, H), jnp.float32),
        mesh=_SC_MESH,
        compiler_params=_SC_PARAMS,
        scratch_types=[
            pltpu.VMEM((UPWD, 2 * UD), jnp.int32),
            pltpu.VMEM((2 * CED, H), jnp.float32),
            pltpu.VMEM((2 * CED, H), jnp.float32),
            pltpu.SemaphoreType.DMA,
            pltpu.SemaphoreType.DMA,
            pltpu.SemaphoreType.DMA,
        ],
    )
    return fn(iv2, T)


def _edge_front_body(ef_ref, w_ref, b_ref, ei_ref, m1_ref):
    p = jnp.dot(ef_ref[...], w_ref[...], preferred_element_type=jnp.float32)
    p = p + b_ref[...]
    ei_ref[...] = p[:, :H]
    m1_ref[...] = jnp.maximum(p[:, H:], 0.0)


def _edge_front(edge_feats, Wcat, bcat):
    grid = (E // EB,)
    return pl.pallas_call(
        _edge_front_body,
        grid=grid,
        in_specs=[
            pl.BlockSpec((EB, D), lambda i: (i, 0)),
            pl.BlockSpec((D, D), lambda i: (0, 0)),
            pl.BlockSpec((1, D), lambda i: (0, 0)),
        ],
        out_specs=[
            pl.BlockSpec((EB, H), lambda i: (i, 0)),
            pl.BlockSpec((EB, H), lambda i: (i, 0)),
        ],
        out_shape=[
            jax.ShapeDtypeStruct((E, H), jnp.float32),
            jax.ShapeDtypeStruct((E, H), jnp.float32),
        ],
    )(edge_feats, Wcat, bcat)


NB = 2000


def _mid_a_body(ss_ref, sd_ref, cs_ref, cd_ref, nf_ref, wg1_ref,
                aei_ref, dinv_ref, xw1_ref, y1_ref):
    cs = cs_ref[0, :, :1] + cs_ref[1, :, :1]
    cd = cd_ref[0, :, :1] + cd_ref[1, :, :1]
    raw_row = (ss_ref[0] + ss_ref[1]) / jnp.maximum(cs, 1.0)
    raw_col = (sd_ref[0] + sd_ref[1]) / jnp.maximum(cd, 1.0)
    aei_ref[...] = _sigmoid(raw_row * raw_col)
    dinv = lax.rsqrt(cd + 1.0)
    dinv_ref[...] = jnp.broadcast_to(dinv, dinv_ref.shape)
    xw1 = jnp.dot(nf_ref[...], wg1_ref[...], preferred_element_type=jnp.float32)
    xw1_ref[...] = xw1
    y1_ref[...] = xw1 * dinv


def _mid_a(sumS, sumD, cS, cD, node_feats, W_g1):
    grid = (NPAD // NB,)
    return pl.pallas_call(
        _mid_a_body,
        grid=grid,
        in_specs=[
            pl.BlockSpec((2, NB, H), lambda i: (0, i, 0)),
            pl.BlockSpec((2, NB, H), lambda i: (0, i, 0)),
            pl.BlockSpec((2, NB, 8), lambda i: (0, i, 0)),
            pl.BlockSpec((2, NB, 8), lambda i: (0, i, 0)),
            pl.BlockSpec((NB, D), lambda i: (i, 0)),
            pl.BlockSpec((D, H), lambda i: (0, 0)),
        ],
        out_specs=[
            pl.BlockSpec((NB, H), lambda i: (i, 0)),
            pl.BlockSpec((NB, 8), lambda i: (i, 0)),
            pl.BlockSpec((NB, H), lambda i: (i, 0)),
            pl.BlockSpec((NB, H), lambda i: (i, 0)),
        ],
        out_shape=[
            jax.ShapeDtypeStruct((NPAD, H), jnp.float32),
            jax.ShapeDtypeStruct((NPAD, 8), jnp.float32),
            jax.ShapeDtypeStruct((NPAD, H), jnp.float32),
            jax.ShapeDtypeStruct((NPAD, H), jnp.float32),
        ],
    )(sumS, sumD, cS, cD, node_feats, W_g1)


def _mid_b_body(agg_ref, xw1_ref, dinv_ref, aei_ref, wg2_ref, bg1_ref,
                xw2_ref, y2a_ref, y2b_ref):
    dinv = dinv_ref[:, :1]
    h1 = dinv * (agg_ref[0] + agg_ref[1]) + dinv * dinv * xw1_ref[...] + bg1_ref[...]
    x1 = jnp.maximum(h1, 0.0) * aei_ref[...]
    xw2 = jnp.dot(x1, wg2_ref[...], preferred_element_type=jnp.float32)
    xw2_ref[...] = xw2
    y2 = xw2 * dinv
    y2a_ref[...] = y2[:, :H]
    y2b_ref[...] = y2[:, H:]


def _mid_b(agg1, xw1, dinv8, aei, W_g2, b_g1):
    grid = (NPAD // NB,)
    return pl.pallas_call(
        _mid_b_body,
        grid=grid,
        in_specs=[
            pl.BlockSpec((2, NB, H), lambda i: (0, i, 0)),
            pl.BlockSpec((NB, H), lambda i: (i, 0)),
            pl.BlockSpec((NB, 8), lambda i: (i, 0)),
            pl.BlockSpec((NB, H), lambda i: (i, 0)),
            pl.BlockSpec((H, D), lambda i: (0, 0)),
            pl.BlockSpec((1, H), lambda i: (0, 0)),
        ],
        out_specs=[
            pl.BlockSpec((NB, D), lambda i: (i, 0)),
            pl.BlockSpec((NB, H), lambda i: (i, 0)),
            pl.BlockSpec((NB, H), lambda i: (i, 0)),
        ],
        out_shape=[
            jax.ShapeDtypeStruct((NPAD, D), jnp.float32),
            jax.ShapeDtypeStruct((NPAD, H), jnp.float32),
            jax.ShapeDtypeStruct((NPAD, H), jnp.float32),
        ],
    )(agg1, xw1, dinv8, aei, W_g2, b_g1)


def _mid_c_body(agga_ref, aggb_ref, xw2_ref, dinv_ref, wna_ref, bna_ref,
                wnrt_ref, wnrb_ref, bnr_ref, bg2_ref,
                x_ref, t_ref):
    dinv = dinv_ref[:, :1]
    aggtot = jnp.concatenate(
        [agga_ref[0] + agga_ref[1], aggb_ref[0] + aggb_ref[1]], axis=-1)
    h2 = dinv * aggtot + dinv * dinv * xw2_ref[...] + bg2_ref[...]
    x = jnp.maximum(h2, 0.0)
    x_ref[...] = x
    ni = jnp.maximum(
        jnp.dot(x, wna_ref[...], preferred_element_type=jnp.float32) + bna_ref[...],
        0.0)
    a = jnp.dot(ni, wnrt_ref[...], preferred_element_type=jnp.float32) + bnr_ref[...]
    b = jnp.dot(ni, wnrb_ref[...], preferred_element_type=jnp.float32)
    t_ref[...] = jnp.concatenate([a, b], axis=-1)


def _mid_c(agg2a, agg2b, xw2, dinv8, W_na, b_na, W_nr_t, W_nr_b, b_nr, b_g2):
    grid = (NPAD // NB,)
    return pl.pallas_call(
        _mid_c_body,
        grid=grid,
        in_specs=[
            pl.BlockSpec((2, NB, H), lambda i: (0, i, 0)),
            pl.BlockSpec((2, NB, H), lambda i: (0, i, 0)),
            pl.BlockSpec((NB, D), lambda i: (i, 0)),
            pl.BlockSpec((NB, 8), lambda i: (i, 0)),
            pl.BlockSpec((D, H), lambda i: (0, 0)),
            pl.BlockSpec((1, H), lambda i: (0, 0)),
            pl.BlockSpec((H, H), lambda i: (0, 0)),
            pl.BlockSpec((H, H), lambda i: (0, 0)),
            pl.BlockSpec((1, H), lambda i: (0, 0)),
            pl.BlockSpec((1, D), lambda i: (0, 0)),
        ],
        out_specs=[
            pl.BlockSpec((NB, D), lambda i: (i, 0)),
            pl.BlockSpec((NB, D), lambda i: (i, 0)),
        ],
        out_shape=[
            jax.ShapeDtypeStruct((NPAD, D), jnp.float32),
            jax.ShapeDtypeStruct((NPAD, D), jnp.float32),
        ],
    )(agg2a, agg2b, xw2, dinv8, W_na, b_na, W_nr_t, W_nr_b, b_nr, b_g2)


def _edge_back_body(m1_ref, g2_ref, wm2_ref, bm2_ref, e_ref):
    g2 = g2_ref[...]
    g = m1_ref[...] * _sigmoid(g2[:, :H] + g2[:, H:])
    e = jnp.dot(g, wm2_ref[...], preferred_element_type=jnp.float32) + bm2_ref[...]
    e_ref[...] = jnp.maximum(e, 0.0)


def _edge_back(m1, g2, W_m2, b_m2):
    grid = (E // EB,)
    return pl.pallas_call(
        _edge_back_body,
        grid=grid,
        in_specs=[
            pl.BlockSpec((EB, H), lambda i: (i, 0)),
            pl.BlockSpec((EB, D), lambda i: (i, 0)),
            pl.BlockSpec((H, D), lambda i: (0, 0)),
            pl.BlockSpec((1, D), lambda i: (0, 0)),
        ],
        out_specs=pl.BlockSpec((EB, D), lambda i: (i, 0)),
        out_shape=jax.ShapeDtypeStruct((E, D), jnp.float32),
    )(m1, g2, W_m2, b_m2)


def kernel(node_feats, edge_feats, edge_index, W_g1, b_g1, W_g2, b_g2,
           W_ea, b_ea, W_na, b_na, W_nr, b_nr, W_m1, b_m1, W_m2, b_m2):
    src2 = edge_index[0].reshape(NUNITS, U)
    dst2 = edge_index[1].reshape(NUNITS, U)
    nf_pad = jnp.pad(node_feats, ((0, NPAD - N), (0, 0)))

    Wcat = jnp.concatenate([W_ea, W_m1], axis=1)
    bcat = jnp.concatenate([b_ea, b_m1])[None, :]
    edge_ind, m1 = _edge_front(edge_feats, Wcat, bcat)

    sumS, sumD, cS, cD = _sc_scatter_counts(src2, dst2, edge_ind)
    aei, dinv8, xw1, y1 = _mid_a(sumS, sumD, cS, cD, nf_pad, W_g1)

    agg1 = _sc_gcn_agg(src2, dst2, y1, H)
    xw2, y2a, y2b = _mid_b(agg1, xw1, dinv8, aei, W_g2, b_g1[None, :])

    agg2a = _sc_gcn_agg(src2, dst2, y2a, H)
    agg2b = _sc_gcn_agg(src2, dst2, y2b, H)
    x, Tcat = _mid_c(agg2a, agg2b, xw2, dinv8, W_na, b_na[None, :],
                     W_nr[:H], W_nr[H:], b_nr[None, :], b_g2[None, :])

    iv2 = jnp.stack([2 * edge_index[0], 2 * edge_index[1] + 1],
                    axis=-1).reshape(NUNITSD, 2 * UD)
    T = Tcat.reshape(2 * NPAD, H)
    g2 = _sc_edge_gather(iv2, T).reshape(E, D)
    e = _edge_back(m1, g2, W_m2, b_m2[None, :])
    return (x[:N], e)
